# baseline (speedup 1.0000x reference)
import functools

import jax
import jax.numpy as jnp
from jax.experimental import pallas as pl
from jax.experimental.pallas import tpu as pltpu

N = 50000
E = 800000
DIN = 128
DE = 16
H = 64
G = 64
NEG_SLOPE = 0.01

N_P = 50176
NODE_BLK = 1024
N_GRID = N_P // NODE_BLK
EDGE_BLK = 3200
E_GRID = E // EDGE_BLK


def _leaky(v):
    return jnp.where(v >= 0, v, NEG_SLOPE * v)


def _node_mlp_body(x_ref, w1, b1, w2, b2, w3, b3, out_ref):
    h = jnp.maximum(jnp.dot(x_ref[...], w1[...],
                            preferred_element_type=jnp.float32) + b1[...], 0.0)
    h = jnp.maximum(jnp.dot(h, w2[...],
                            preferred_element_type=jnp.float32) + b2[...], 0.0)
    out_ref[...] = jnp.dot(h, w3[...],
                           preferred_element_type=jnp.float32) + b3[...]


def _node_mlp(x_p, p):
    full = lambda shape: pl.BlockSpec(shape, lambda i: (0,) * len(shape))
    return pl.pallas_call(
        _node_mlp_body,
        grid=(N_GRID,),
        in_specs=[
            pl.BlockSpec((NODE_BLK, DIN), lambda i: (i, 0)),
            full((DIN, H)), full((1, H)),
            full((H, H)), full((1, H)),
            full((H, H)), full((1, H)),
        ],
        out_specs=pl.BlockSpec((NODE_BLK, H), lambda i: (i, 0)),
        out_shape=jax.ShapeDtypeStruct((N_P, H), jnp.float32),
    )(x_p, p['Wnx1'], p['bnx1'].reshape(1, H),
      p['Wnx2'], p['bnx2'].reshape(1, H),
      p['Wnx3'], p['bnx3'].reshape(1, H))


def _edge_mlp_body(ea_ref, w1, b1, w2, b2, w3, b3, we, be, out_ref):
    t = jnp.maximum(jnp.dot(ea_ref[...], w1[...],
                            preferred_element_type=jnp.float32) + b1[...], 0.0)
    t = jnp.maximum(jnp.dot(t, w2[...],
                            preferred_element_type=jnp.float32) + b2[...], 0.0)
    w3e = jnp.dot(w3[...], we[...], preferred_element_type=jnp.float32)
    b3e = jnp.dot(b3[...], we[...], preferred_element_type=jnp.float32) + be[...]
    out_ref[...] = jnp.dot(t, w3e, preferred_element_type=jnp.float32) + b3e


def _edge_mlp(edge_attr, p):
    full = lambda shape: pl.BlockSpec(shape, lambda i: (0,) * len(shape))
    return pl.pallas_call(
        _edge_mlp_body,
        grid=(E_GRID,),
        in_specs=[
            pl.BlockSpec((EDGE_BLK, DE), lambda i: (i, 0)),
            full((DE, H)), full((1, H)),
            full((H, H)), full((1, H)),
            full((H, H)), full((1, H)),
            full((H, H)), full((1, H)),
        ],
        out_specs=pl.BlockSpec((EDGE_BLK, H), lambda i: (i, 0)),
        out_shape=jax.ShapeDtypeStruct((E, H), jnp.float32),
    )(edge_attr, p['Wne1'], p['bne1'].reshape(1, H),
      p['Wne2'], p['bne2'].reshape(1, H),
      p['Wne3'], p['bne3'].reshape(1, H),
      p['We'], p['be'].reshape(1, H))


def _head_body(h_ref, agg_ref, ids_ref, eps_ref, wg1, bg1, wg2, bg2,
               wo1, bo1, gamma, beta, wo2, bo2,
               o_ref, sig_ref, hp_ref):
    step = pl.program_id(0)

    @pl.when(step == 0)
    def _init():
        hp_ref[...] = jnp.full((G, H), -1e30, jnp.float32)

    z = (1.0 + eps_ref[0, 0]) * h_ref[...] + agg_ref[...]
    z = _leaky(jnp.dot(z, wg1[...], preferred_element_type=jnp.float32)
               + bg1[...])
    z2 = jnp.dot(z, wg2[...], preferred_element_type=jnp.float32) + bg2[...]

    ids = ids_ref[...]
    gmin = jnp.min(ids)
    gmax = jnp.minimum(jnp.max(ids), G - 1)

    def body(g, _):
        mask = ids == g
        m = jnp.max(jnp.where(mask, z2, -1e30), axis=0, keepdims=True)
        cur = hp_ref[pl.ds(g, 1), :]
        hp_ref[pl.ds(g, 1), :] = jnp.maximum(cur, m)
        return 0

    jax.lax.fori_loop(gmin, gmax + 1, body, 0)

    @pl.when(step == N_GRID - 1)
    def _head():
        hp = hp_ref[...]
        hp = jnp.where(hp < -1e29, 0.0, hp)
        o = jnp.dot(hp, wo1[...], preferred_element_type=jnp.float32) + bo1[...]
        o = o * (1.0 / jnp.sqrt(1.0 + 1e-5)) * gamma[...] + beta[...]
        o = _leaky(o)
        o2 = jnp.dot(o, wo2[...], preferred_element_type=jnp.float32) + bo2[0, 0]
        o_ref[...] = o2
        sig_ref[...] = 1.0 / (1.0 + jnp.exp(-o2))


def _head(h_p, agg_p, batch2, p):
    full = lambda shape: pl.BlockSpec(shape, lambda i: (0,) * len(shape))
    return pl.pallas_call(
        _head_body,
        grid=(N_GRID,),
        in_specs=[
            pl.BlockSpec((NODE_BLK, H), lambda i: (i, 0)),
            pl.BlockSpec((NODE_BLK, H), lambda i: (i, 0)),
            pl.BlockSpec((NODE_BLK, 1), lambda i: (i, 0)),
            full((1, 1)),
            full((H, H)), full((1, H)),
            full((H, H)), full((1, H)),
            full((H, H)), full((1, H)),
            full((1, H)), full((1, H)),
            full((H, 1)), full((1, 1)),
        ],
        out_specs=[full((G, 1)), full((G, 1))],
        out_shape=[jax.ShapeDtypeStruct((G, 1), jnp.float32),
                   jax.ShapeDtypeStruct((G, 1), jnp.float32)],
        scratch_shapes=[pltpu.VMEM((G, H), jnp.float32)],
    )(h_p, agg_p, batch2, p['eps'].reshape(1, 1),
      p['Wg1'], p['bg1'].reshape(1, H),
      p['Wg2'], p['bg2'].reshape(1, H),
      p['Wo1'], p['bo1'].reshape(1, H),
      p['gamma'].reshape(1, H), p['beta'].reshape(1, H),
      p['Wo2'], p['bo2'].reshape(1, 1))


def kernel(x, edge_index, batch, edge_attr, params):
    p = params
    x_p = jnp.pad(x, ((0, N_P - N), (0, 0)))
    batch_p = jnp.pad(batch, (0, N_P - N), constant_values=G)
    batch2 = batch_p.reshape(N_P, 1)

    h_p = _node_mlp(x_p, p)
    eaw = _edge_mlp(edge_attr, p)

    src = edge_index[0]
    dst = edge_index[1]
    msg = jnp.maximum(jnp.take(h_p[:N], src, axis=0) + eaw, 0.0)
    agg = jax.ops.segment_max(msg, dst, num_segments=N)
    agg = jnp.where(jnp.isfinite(agg), agg, 0.0)
    agg_p = jnp.pad(agg, ((0, N_P - N), (0, 0)))

    o, sig = _head(h_p, agg_p, batch2, p)
    return (o, sig)


# speedup vs baseline: 1.2085x; 1.2085x over previous
import functools

import jax
import jax.numpy as jnp
from jax import lax
from jax.experimental import pallas as pl
from jax.experimental.pallas import tpu as pltpu
from jax.experimental.pallas import tpu_sc as plsc

N = 50000
E = 800000
DIN = 128
DE = 16
H = 64
G = 64
NEG_SLOPE = 0.01

N_P = 50176
HP = 128
NODE_BLK = 1024
N_GRID = N_P // NODE_BLK
EDGE_BLK = 3584
E_GRID = 802816 // EDGE_BLK


def _leaky(v):
    return jnp.where(v >= 0, v, NEG_SLOPE * v)


def _node_mlp_body(x_ref, w1, b1, w2, b2, w3, b3, out_ref):
    h = jnp.maximum(jnp.dot(x_ref[...], w1[...],
                            preferred_element_type=jnp.float32) + b1[...], 0.0)
    h = jnp.maximum(jnp.dot(h, w2[...],
                            preferred_element_type=jnp.float32) + b2[...], 0.0)
    out_ref[...] = jnp.dot(h, w3[...],
                           preferred_element_type=jnp.float32) + b3[...]


def _node_mlp(x_p, p):
    full = lambda shape: pl.BlockSpec(shape, lambda i: (0,) * len(shape))
    return pl.pallas_call(
        _node_mlp_body,
        grid=(N_GRID,),
        in_specs=[
            pl.BlockSpec((NODE_BLK, DIN), lambda i: (i, 0)),
            full((DIN, H)), full((1, H)),
            full((H, H)), full((1, H)),
            full((H, HP)), full((1, HP)),
        ],
        out_specs=pl.BlockSpec((NODE_BLK, HP), lambda i: (i, 0)),
        out_shape=jax.ShapeDtypeStruct((N_P, HP), jnp.float32),
    )(x_p, p['Wnx1'], p['bnx1'].reshape(1, H),
      p['Wnx2'], p['bnx2'].reshape(1, H),
      jnp.pad(p['Wnx3'], ((0, 0), (0, HP - H))),
      jnp.pad(p['bnx3'], (0, HP - H)).reshape(1, HP))


def _edge_mlp_body(ea_ref, w1, b1, w2, b2, w3, b3, we, be, out_ref):
    t = jnp.maximum(jnp.dot(ea_ref[...], w1[...],
                            preferred_element_type=jnp.float32) + b1[...], 0.0)
    t = jnp.maximum(jnp.dot(t, w2[...],
                            preferred_element_type=jnp.float32) + b2[...], 0.0)
    w3e = jnp.dot(w3[...], we[...], preferred_element_type=jnp.float32)
    b3e = jnp.dot(b3[...], we[...], preferred_element_type=jnp.float32) + be[...]
    out_ref[...] = jnp.dot(t, w3e, preferred_element_type=jnp.float32) + b3e


def _edge_mlp(edge_attr, p):
    full = lambda shape: pl.BlockSpec(shape, lambda i: (0,) * len(shape))
    return pl.pallas_call(
        _edge_mlp_body,
        grid=(E_GRID,),
        in_specs=[
            pl.BlockSpec((EDGE_BLK, DE), lambda i: (i, 0)),
            full((DE, H)), full((1, H)),
            full((H, H)), full((1, H)),
            full((H, H)), full((1, H)),
            full((H, HP)), full((1, HP)),
        ],
        out_specs=pl.BlockSpec((EDGE_BLK, HP), lambda i: (i, 0)),
        out_shape=jax.ShapeDtypeStruct((E_P, HP), jnp.float32),
    )(edge_attr, p['Wne1'], p['bne1'].reshape(1, H),
      p['Wne2'], p['bne2'].reshape(1, H),
      p['Wne3'], p['bne3'].reshape(1, H),
      jnp.pad(p['We'], ((0, 0), (0, HP - H))),
      jnp.pad(p['be'], (0, HP - H)).reshape(1, HP))


def _head_body(h_ref, agg_ref, ids_ref, eps_ref, wg1, bg1, wg2, bg2,
               wo1, bo1, gamma, beta, wo2, bo2,
               o_ref, sig_ref, hp_ref):
    step = pl.program_id(0)

    @pl.when(step == 0)
    def _init():
        hp_ref[...] = jnp.full((G, H), -1e30, jnp.float32)

    z = (1.0 + eps_ref[0, 0]) * h_ref[:, :H] + agg_ref[...]
    z = _leaky(jnp.dot(z, wg1[...], preferred_element_type=jnp.float32)
               + bg1[...])
    z2 = jnp.dot(z, wg2[...], preferred_element_type=jnp.float32) + bg2[...]

    ids = ids_ref[...]
    gmin = jnp.min(ids)
    gmax = jnp.minimum(jnp.max(ids), G - 1)

    def body(g, _):
        mask = ids == g
        m = jnp.max(jnp.where(mask, z2, -1e30), axis=0, keepdims=True)
        cur = hp_ref[pl.ds(g, 1), :]
        hp_ref[pl.ds(g, 1), :] = jnp.maximum(cur, m)
        return 0

    jax.lax.fori_loop(gmin, gmax + 1, body, 0)

    @pl.when(step == N_GRID - 1)
    def _head():
        hp = hp_ref[...]
        hp = jnp.where(hp < -1e29, 0.0, hp)
        o = jnp.dot(hp, wo1[...], preferred_element_type=jnp.float32) + bo1[...]
        o = o * (1.0 / jnp.sqrt(1.0 + 1e-5)) * gamma[...] + beta[...]
        o = _leaky(o)
        o2 = jnp.dot(o, wo2[...], preferred_element_type=jnp.float32) + bo2[0, 0]
        o_ref[...] = o2
        sig_ref[...] = 1.0 / (1.0 + jnp.exp(-o2))


def _head(h_p, agg_p, batch2, p):
    full = lambda shape: pl.BlockSpec(shape, lambda i: (0,) * len(shape))
    return pl.pallas_call(
        _head_body,
        grid=(N_GRID,),
        in_specs=[
            pl.BlockSpec((NODE_BLK, HP), lambda i: (i, 0)),
            pl.BlockSpec((NODE_BLK, H), lambda i: (i, 0)),
            pl.BlockSpec((NODE_BLK, 1), lambda i: (i, 0)),
            full((1, 1)),
            full((H, H)), full((1, H)),
            full((H, H)), full((1, H)),
            full((H, H)), full((1, H)),
            full((1, H)), full((1, H)),
            full((H, 1)), full((1, 1)),
        ],
        out_specs=[full((G, 1)), full((G, 1))],
        out_shape=[jax.ShapeDtypeStruct((G, 1), jnp.float32),
                   jax.ShapeDtypeStruct((G, 1), jnp.float32)],
        scratch_shapes=[pltpu.VMEM((G, H), jnp.float32)],
    )(h_p, agg_p, batch2, p['eps'].reshape(1, 1),
      p['Wg1'], p['bg1'].reshape(1, H),
      p['Wg2'], p['bg2'].reshape(1, H),
      p['Wo1'], p['bo1'].reshape(1, H),
      p['gamma'].reshape(1, H), p['beta'].reshape(1, H),
      p['Wo2'], p['bo2'].reshape(1, 1))


NW = 32
NPASS = 2
R = N_P // (NW * NPASS)
TRASH = R
SC_CHUNK = 2048
N_GROUPS = SC_CHUNK // 16
E_P = 802816
N_CHUNKS = E_P // SC_CHUNK
BATCH = 128
CAP = BATCH + SC_CHUNK + 16


def _sc_agg_body(h_hbm, eaw_hbm, src_hbm, dst_hbm, out_hbm,
                 dstv, srcv, svacc, pkacc, idbuf, dlbuf, pbuf, hrows, erows,
                 agg, sem1, sem2):
    cid = lax.axis_index("c")
    sid = lax.axis_index("s")
    wid = sid * 2 + cid
    iota = lax.iota(jnp.int32, 16)
    zero16 = jnp.zeros((16,), jnp.float32)
    pbuf[pl.ds(0, 16)] = jnp.zeros((16,), jnp.int32)

    def apply_batch(lo):
        def unpack_body(g, _):
            v = pkacc[pl.ds(g * 16, 16)]
            idbuf[pl.ds(g * 16, 16)] = v & 0xFFFFF
            dlbuf[pl.ds(g * 16, 16)] = jax.lax.shift_right_logical(v, 20)
            return 0

        lax.fori_loop(0, BATCH // 16, unpack_body, 0)

        cp1 = pltpu.async_copy(h_hbm.at[svacc.at[pl.ds(0, BATCH)]],
                               hrows, sem1)
        cp2 = pltpu.async_copy(eaw_hbm.at[idbuf], erows, sem2)
        cp1.wait()
        cp2.wait()

        def edge_grp_body(g, _):
            dlv = dlbuf[pl.ds(g * 16, 16)] * H
            for lane in range(16):
                i = g * 16 + lane
                rb = dlv[lane]
                for q in range(4):
                    hv = hrows[i, pl.ds(q * 16, 16)]
                    ev = erows[i, pl.ds(q * 16, 16)]
                    msg = jnp.maximum(hv + ev, 0.0)
                    cur = agg[pl.ds(rb + q * 16, 16)]
                    agg[pl.ds(rb + q * 16, 16)] = jnp.maximum(cur, msg)
            return 0

        lax.fori_loop(0, BATCH // 16, edge_grp_body, 0)

    def shift_batch():
        def shift_body(j, _):
            for ref in (svacc, pkacc):
                ref[pl.ds(j * 16, 16)] = ref[pl.ds(BATCH + j * 16, 16)]
            return 0

        lax.fori_loop(0, SC_CHUNK // 16, shift_body, 0)

    def pass_body(p, _):
        lo = (p * NW + wid) * R
        hi = lo + R
        lo_vec = jnp.full((16,), lo, jnp.int32)
        hi_vec = jnp.full((16,), hi, jnp.int32)

        def zero_body(r, _):
            for q in range(4):
                agg[pl.ds(r * H + q * 16, 16)] = zero16
            return 0

        lax.fori_loop(0, R + 1, zero_body, 0)

        def chunk_body(c, cnt):
            base = c * SC_CHUNK
            pltpu.sync_copy(dst_hbm.at[pl.ds(base, SC_CHUNK)], dstv)
            pltpu.sync_copy(src_hbm.at[pl.ds(base, SC_CHUNK)], srcv)
            base_vec = jnp.full((16,), base, jnp.int32) + iota

            def count_body(g, qc):
                d = dstv[pl.ds(g * 16, 16)]
                m = (d >= lo_vec) & (d < hi_vec)
                return qc + jnp.where(m, 1, 0).astype(jnp.int32)

            qc = lax.fori_loop(0, N_GROUPS, count_body,
                               jnp.zeros((16,), jnp.int32))

            s = qc
            for sh in (1, 2, 4, 8):
                pbuf[pl.ds(16, 16)] = s
                s = s + plsc.load_gather(pbuf, [iota + (16 - sh)])
            excl = s - qc
            total = s[15]

            def fill_body(g, wp):
                d = dstv[pl.ds(g * 16, 16)]
                sv = srcv[pl.ds(g * 16, 16)]
                m = (d >= lo_vec) & (d < hi_vec)
                dl = d - lo_vec
                packed = (base_vec + g * 16) | jax.lax.shift_left(dl, 20)
                dest = jnp.where(m, wp, CAP - 16 + iota)
                plsc.store_scatter(svacc, [dest], sv)
                plsc.store_scatter(pkacc, [dest], packed)
                return wp + jnp.where(m, 1, 0).astype(jnp.int32)

            lax.fori_loop(0, N_GROUPS, fill_body,
                          jnp.full((16,), cnt, jnp.int32) + excl)
            cnt = cnt + total

            def drain_cond(cc):
                return cc >= BATCH

            def drain_body(cc):
                apply_batch(lo)
                shift_batch()
                return cc - BATCH

            cnt = lax.while_loop(drain_cond, drain_body, cnt)
            return cnt

        cnt = lax.fori_loop(0, N_CHUNKS, chunk_body, jnp.int32(0))

        pad_pk = (jnp.full((16,), TRASH << 20, jnp.int32)
                  | (wid * SC_CHUNK + iota))
        for j in range(BATCH // 16):
            dest = jnp.full((16,), cnt, jnp.int32) + iota + j * 16
            plsc.store_scatter(svacc, [dest], lo_vec + iota)
            plsc.store_scatter(pkacc, [dest], pad_pk + j * 16)
        apply_batch(lo)

        pltpu.sync_copy(agg.at[pl.ds(0, R * H)],
                        out_hbm.at[pl.ds(lo * H, R * H)])
        return 0

    lax.fori_loop(0, NPASS, pass_body, 0)


def _sc_agg(h_p, eaw, src_p, dst_p):
    mesh = plsc.VectorSubcoreMesh(core_axis_name="c", subcore_axis_name="s")
    f = pl.kernel(
        _sc_agg_body,
        out_type=jax.ShapeDtypeStruct((N_P * H,), jnp.float32),
        mesh=mesh,
        compiler_params=pltpu.CompilerParams(needs_layout_passes=False),
        scratch_types=[
            pltpu.VMEM((SC_CHUNK,), jnp.int32),
            pltpu.VMEM((SC_CHUNK,), jnp.int32),
            pltpu.VMEM((CAP,), jnp.int32),
            pltpu.VMEM((CAP,), jnp.int32),
            pltpu.VMEM((BATCH,), jnp.int32),
            pltpu.VMEM((BATCH,), jnp.int32),
            pltpu.VMEM((32,), jnp.int32),
            pltpu.VMEM((BATCH, HP), jnp.float32),
            pltpu.VMEM((BATCH, HP), jnp.float32),
            pltpu.VMEM(((R + 1) * H,), jnp.float32),
            pltpu.SemaphoreType.DMA,
            pltpu.SemaphoreType.DMA,
        ],
    )
    return f(h_p, eaw, src_p, dst_p)


def kernel(x, edge_index, batch, edge_attr, params):
    p = params
    x_p = jnp.pad(x, ((0, N_P - N), (0, 0)))
    batch_p = jnp.pad(batch, (0, N_P - N), constant_values=G)
    batch2 = batch_p.reshape(N_P, 1)

    ea_p = jnp.pad(edge_attr, ((0, E_P - E), (0, 0)))
    src_p = jnp.pad(edge_index[0], (0, E_P - E))
    dst_p = jnp.pad(edge_index[1], (0, E_P - E), constant_values=N_P - 1)

    h_p = _node_mlp(x_p, p)
    eaw = _edge_mlp(ea_p, p)
    agg_p = _sc_agg(h_p, eaw, src_p, dst_p).reshape(N_P, H)

    o, sig = _head(h_p, agg_p, batch2, p)
    return (o, sig)


# speedup vs baseline: 1.6053x; 1.3283x over previous
import functools

import jax
import jax.numpy as jnp
from jax import lax
from jax.experimental import pallas as pl
from jax.experimental.pallas import tpu as pltpu
from jax.experimental.pallas import tpu_sc as plsc

N = 50000
E = 800000
DIN = 128
DE = 16
H = 64
G = 64
NEG_SLOPE = 0.01

N_P = 50176
HP = 128
NODE_BLK = 1024
N_GRID = N_P // NODE_BLK
EDGE_BLK = 3584
E_GRID = 802816 // EDGE_BLK


def _leaky(v):
    return jnp.where(v >= 0, v, NEG_SLOPE * v)


def _node_mlp_body(x_ref, w1, b1, w2, b2, w3, b3, out_ref):
    h = jnp.maximum(jnp.dot(x_ref[...], w1[...],
                            preferred_element_type=jnp.float32) + b1[...], 0.0)
    h = jnp.maximum(jnp.dot(h, w2[...],
                            preferred_element_type=jnp.float32) + b2[...], 0.0)
    out_ref[...] = jnp.dot(h, w3[...],
                           preferred_element_type=jnp.float32) + b3[...]


def _node_mlp(x_p, p):
    full = lambda shape: pl.BlockSpec(shape, lambda i: (0,) * len(shape))
    return pl.pallas_call(
        _node_mlp_body,
        grid=(N_GRID,),
        in_specs=[
            pl.BlockSpec((NODE_BLK, DIN), lambda i: (i, 0)),
            full((DIN, H)), full((1, H)),
            full((H, H)), full((1, H)),
            full((H, HP)), full((1, HP)),
        ],
        out_specs=pl.BlockSpec((NODE_BLK, HP), lambda i: (i, 0)),
        out_shape=jax.ShapeDtypeStruct((N_P, HP), jnp.float32),
    )(x_p, p['Wnx1'], p['bnx1'].reshape(1, H),
      p['Wnx2'], p['bnx2'].reshape(1, H),
      jnp.pad(p['Wnx3'], ((0, 0), (0, HP - H))),
      jnp.pad(p['bnx3'], (0, HP - H)).reshape(1, HP))


def _edge_mlp_body(ea_ref, w1, b1, w2, b2, w3, b3, we, be, out_ref):
    t = jnp.maximum(jnp.dot(ea_ref[...], w1[...],
                            preferred_element_type=jnp.float32) + b1[...], 0.0)
    t = jnp.maximum(jnp.dot(t, w2[...],
                            preferred_element_type=jnp.float32) + b2[...], 0.0)
    w3e = jnp.dot(w3[...], we[...], preferred_element_type=jnp.float32)
    b3e = jnp.dot(b3[...], we[...], preferred_element_type=jnp.float32) + be[...]
    out_ref[...] = jnp.dot(t, w3e, preferred_element_type=jnp.float32) + b3e


def _edge_mlp(edge_attr, p):
    full = lambda shape: pl.BlockSpec(shape, lambda i: (0,) * len(shape))
    return pl.pallas_call(
        _edge_mlp_body,
        grid=(E_GRID,),
        in_specs=[
            pl.BlockSpec((EDGE_BLK, DE), lambda i: (i, 0)),
            full((DE, H)), full((1, H)),
            full((H, H)), full((1, H)),
            full((H, H)), full((1, H)),
            full((H, HP)), full((1, HP)),
        ],
        out_specs=pl.BlockSpec((EDGE_BLK, HP), lambda i: (i, 0)),
        out_shape=jax.ShapeDtypeStruct((E_P, HP), jnp.float32),
    )(edge_attr, p['Wne1'], p['bne1'].reshape(1, H),
      p['Wne2'], p['bne2'].reshape(1, H),
      p['Wne3'], p['bne3'].reshape(1, H),
      jnp.pad(p['We'], ((0, 0), (0, HP - H))),
      jnp.pad(p['be'], (0, HP - H)).reshape(1, HP))


def _head_body(h_ref, agg_ref, ids_ref, eps_ref, wg1, bg1, wg2, bg2,
               wo1, bo1, gamma, beta, wo2, bo2,
               o_ref, sig_ref, hp_ref):
    step = pl.program_id(0)

    @pl.when(step == 0)
    def _init():
        hp_ref[...] = jnp.full((G, H), -1e30, jnp.float32)

    z = (1.0 + eps_ref[0, 0]) * h_ref[:, :H] + agg_ref[...]
    z = _leaky(jnp.dot(z, wg1[...], preferred_element_type=jnp.float32)
               + bg1[...])
    z2 = jnp.dot(z, wg2[...], preferred_element_type=jnp.float32) + bg2[...]

    ids = ids_ref[...]
    gmin = jnp.min(ids)
    gmax = jnp.minimum(jnp.max(ids), G - 1)

    def body(g, _):
        mask = ids == g
        m = jnp.max(jnp.where(mask, z2, -1e30), axis=0, keepdims=True)
        cur = hp_ref[pl.ds(g, 1), :]
        hp_ref[pl.ds(g, 1), :] = jnp.maximum(cur, m)
        return 0

    jax.lax.fori_loop(gmin, gmax + 1, body, 0)

    @pl.when(step == N_GRID - 1)
    def _head():
        hp = hp_ref[...]
        hp = jnp.where(hp < -1e29, 0.0, hp)
        o = jnp.dot(hp, wo1[...], preferred_element_type=jnp.float32) + bo1[...]
        o = o * (1.0 / jnp.sqrt(1.0 + 1e-5)) * gamma[...] + beta[...]
        o = _leaky(o)
        o2 = jnp.dot(o, wo2[...], preferred_element_type=jnp.float32) + bo2[0, 0]
        o_ref[...] = o2
        sig_ref[...] = 1.0 / (1.0 + jnp.exp(-o2))


def _head(h_p, agg_p, batch2, p):
    full = lambda shape: pl.BlockSpec(shape, lambda i: (0,) * len(shape))
    return pl.pallas_call(
        _head_body,
        grid=(N_GRID,),
        in_specs=[
            pl.BlockSpec((NODE_BLK, HP), lambda i: (i, 0)),
            pl.BlockSpec((NODE_BLK, H), lambda i: (i, 0)),
            pl.BlockSpec((NODE_BLK, 1), lambda i: (i, 0)),
            full((1, 1)),
            full((H, H)), full((1, H)),
            full((H, H)), full((1, H)),
            full((H, H)), full((1, H)),
            full((1, H)), full((1, H)),
            full((H, 1)), full((1, 1)),
        ],
        out_specs=[full((G, 1)), full((G, 1))],
        out_shape=[jax.ShapeDtypeStruct((G, 1), jnp.float32),
                   jax.ShapeDtypeStruct((G, 1), jnp.float32)],
        scratch_shapes=[pltpu.VMEM((G, H), jnp.float32)],
    )(h_p, agg_p, batch2, p['eps'].reshape(1, 1),
      p['Wg1'], p['bg1'].reshape(1, H),
      p['Wg2'], p['bg2'].reshape(1, H),
      p['Wo1'], p['bo1'].reshape(1, H),
      p['gamma'].reshape(1, H), p['beta'].reshape(1, H),
      p['Wo2'], p['bo2'].reshape(1, 1))


NW = 32
NPASS = 2
R = N_P // (NW * NPASS)
TRASH = R
SC_CHUNK = 4096
N_GROUPS = SC_CHUNK // 16
E_P = 802816
N_CHUNKS = E_P // SC_CHUNK
BATCH = 128
CAP = BATCH + SC_CHUNK + 16


def _sc_agg_body(h_hbm, eaw_hbm, src_hbm, dst_hbm, out_hbm,
                 dstv0, dstv1, srcv0, srcv1, svacc, pkacc, idbuf, dlbuf,
                 pbuf, hrows, erows, agg, sem1, sem2,
                 sd0, sd1, ss0, ss1):
    cid = lax.axis_index("c")
    sid = lax.axis_index("s")
    wid = sid * 2 + cid
    iota = lax.iota(jnp.int32, 16)
    zero16 = jnp.zeros((16,), jnp.float32)
    pbuf[pl.ds(0, 16)] = jnp.zeros((16,), jnp.int32)

    def apply_batch(lo):
        def unpack_body(g, _):
            v = pkacc[pl.ds(g * 16, 16)]
            idbuf[pl.ds(g * 16, 16)] = v & 0xFFFFF
            dlbuf[pl.ds(g * 16, 16)] = jax.lax.shift_right_logical(v, 20)
            return 0

        lax.fori_loop(0, BATCH // 16, unpack_body, 0)

        cp1 = pltpu.async_copy(h_hbm.at[svacc.at[pl.ds(0, BATCH)]],
                               hrows, sem1)
        cp2 = pltpu.async_copy(eaw_hbm.at[idbuf], erows, sem2)
        cp1.wait()
        cp2.wait()

        def edge_grp_body(g, _):
            dlv = dlbuf[pl.ds(g * 16, 16)] * H
            for lane in range(16):
                i = g * 16 + lane
                rb = dlv[lane]
                for q in range(4):
                    hv = hrows[i, pl.ds(q * 16, 16)]
                    ev = erows[i, pl.ds(q * 16, 16)]
                    msg = jnp.maximum(hv + ev, 0.0)
                    cur = agg[pl.ds(rb + q * 16, 16)]
                    agg[pl.ds(rb + q * 16, 16)] = jnp.maximum(cur, msg)
            return 0

        lax.fori_loop(0, BATCH // 16, edge_grp_body, 0)

    def shift_batch():
        def shift_body(j, _):
            for ref in (svacc, pkacc):
                ref[pl.ds(j * 16, 16)] = ref[pl.ds(BATCH + j * 16, 16)]
            return 0

        lax.fori_loop(0, SC_CHUNK // 16, shift_body, 0)

    def pass_body(p, _):
        lo = (p * NW + wid) * R
        hi = lo + R
        lo_vec = jnp.full((16,), lo, jnp.int32)
        hi_vec = jnp.full((16,), hi, jnp.int32)

        def zero_body(r, _):
            for q in range(4):
                agg[pl.ds(r * H + q * 16, 16)] = zero16
            return 0

        lax.fori_loop(0, R + 1, zero_body, 0)

        def start_stage(c, db, sb, semd, sems):
            base = c * SC_CHUNK
            pltpu.async_copy(dst_hbm.at[pl.ds(base, SC_CHUNK)], db, semd)
            pltpu.async_copy(src_hbm.at[pl.ds(base, SC_CHUNK)], sb, sems)

        start_stage(0, dstv0, srcv0, sd0, ss0)
        start_stage(1, dstv1, srcv1, sd1, ss1)

        def chunk_pair_body(cc, cnt):
            for b, (db, sb, semd, sems) in enumerate(
                    ((dstv0, srcv0, sd0, ss0), (dstv1, srcv1, sd1, ss1))):
                c = cc * 2 + b
                base = c * SC_CHUNK
                pltpu.make_async_copy(dst_hbm.at[pl.ds(0, SC_CHUNK)],
                                      db, semd).wait()
                pltpu.make_async_copy(src_hbm.at[pl.ds(0, SC_CHUNK)],
                                      sb, sems).wait()
                base_vec = jnp.full((16,), base, jnp.int32) + iota

                def count_body(g, qc):
                    d = db[pl.ds(g * 16, 16)]
                    m = (d >= lo_vec) & (d < hi_vec)
                    return qc + jnp.where(m, 1, 0).astype(jnp.int32)

                qc = lax.fori_loop(0, N_GROUPS, count_body,
                                   jnp.zeros((16,), jnp.int32), unroll=8)

                s = qc
                for sh in (1, 2, 4, 8):
                    pbuf[pl.ds(16, 16)] = s
                    s = s + plsc.load_gather(pbuf, [iota + (16 - sh)])
                excl = s - qc
                total = s[15]

                def fill_body(g, wp):
                    d = db[pl.ds(g * 16, 16)]
                    sv = sb[pl.ds(g * 16, 16)]
                    m = (d >= lo_vec) & (d < hi_vec)
                    dl = d - lo_vec
                    packed = (base_vec + g * 16) | jax.lax.shift_left(dl, 20)
                    dest = jnp.where(m, wp, CAP - 16 + iota)
                    plsc.store_scatter(svacc, [dest], sv)
                    plsc.store_scatter(pkacc, [dest], packed)
                    return wp + jnp.where(m, 1, 0).astype(jnp.int32)

                lax.fori_loop(0, N_GROUPS, fill_body,
                              jnp.full((16,), cnt, jnp.int32) + excl,
                              unroll=4)
                cnt = cnt + total

                @pl.when(c + 2 < N_CHUNKS)
                def _prefetch():
                    start_stage(c + 2, db, sb, semd, sems)

                def drain_cond(cc2):
                    return cc2 >= BATCH

                def drain_body(cc2):
                    apply_batch(lo)
                    shift_batch()
                    return cc2 - BATCH

                cnt = lax.while_loop(drain_cond, drain_body, cnt)
            return cnt

        cnt = lax.fori_loop(0, N_CHUNKS // 2, chunk_pair_body, jnp.int32(0))

        pad_pk = (jnp.full((16,), TRASH << 20, jnp.int32)
                  | (wid * SC_CHUNK + iota))
        for j in range(BATCH // 16):
            dest = jnp.full((16,), cnt, jnp.int32) + iota + j * 16
            plsc.store_scatter(svacc, [dest], lo_vec + iota)
            plsc.store_scatter(pkacc, [dest], pad_pk + j * 16)
        apply_batch(lo)

        pltpu.sync_copy(agg.at[pl.ds(0, R * H)],
                        out_hbm.at[pl.ds(lo * H, R * H)])
        return 0

    lax.fori_loop(0, NPASS, pass_body, 0)


def _sc_agg(h_p, eaw, src_p, dst_p):
    mesh = plsc.VectorSubcoreMesh(core_axis_name="c", subcore_axis_name="s")
    f = pl.kernel(
        _sc_agg_body,
        out_type=jax.ShapeDtypeStruct((N_P * H,), jnp.float32),
        mesh=mesh,
        compiler_params=pltpu.CompilerParams(needs_layout_passes=False),
        scratch_types=[
            pltpu.VMEM((SC_CHUNK,), jnp.int32),
            pltpu.VMEM((SC_CHUNK,), jnp.int32),
            pltpu.VMEM((SC_CHUNK,), jnp.int32),
            pltpu.VMEM((SC_CHUNK,), jnp.int32),
            pltpu.VMEM((CAP,), jnp.int32),
            pltpu.VMEM((CAP,), jnp.int32),
            pltpu.VMEM((BATCH,), jnp.int32),
            pltpu.VMEM((BATCH,), jnp.int32),
            pltpu.VMEM((32,), jnp.int32),
            pltpu.VMEM((BATCH, HP), jnp.float32),
            pltpu.VMEM((BATCH, HP), jnp.float32),
            pltpu.VMEM(((R + 1) * H,), jnp.float32),
            pltpu.SemaphoreType.DMA,
            pltpu.SemaphoreType.DMA,
            pltpu.SemaphoreType.DMA,
            pltpu.SemaphoreType.DMA,
            pltpu.SemaphoreType.DMA,
            pltpu.SemaphoreType.DMA,
        ],
    )
    return f(h_p, eaw, src_p, dst_p)


def kernel(x, edge_index, batch, edge_attr, params):
    p = params
    x_p = jnp.pad(x, ((0, N_P - N), (0, 0)))
    batch_p = jnp.pad(batch, (0, N_P - N), constant_values=G)
    batch2 = batch_p.reshape(N_P, 1)

    ea_p = jnp.pad(edge_attr, ((0, E_P - E), (0, 0)))
    src_p = jnp.pad(edge_index[0], (0, E_P - E))
    dst_p = jnp.pad(edge_index[1], (0, E_P - E), constant_values=N_P - 1)

    h_p = _node_mlp(x_p, p)
    eaw = _edge_mlp(ea_p, p)
    agg_p = _sc_agg(h_p, eaw, src_p, dst_p).reshape(N_P, H)

    o, sig = _head(h_p, agg_p, batch2, p)
    return (o, sig)


# speedup vs baseline: 1.9089x; 1.1892x over previous
import functools

import jax
import jax.numpy as jnp
from jax import lax
from jax.experimental import pallas as pl
from jax.experimental.pallas import tpu as pltpu
from jax.experimental.pallas import tpu_sc as plsc

N = 50000
E = 800000
DIN = 128
DE = 16
H = 64
G = 64
NEG_SLOPE = 0.01

N_P = 50176
HP = 128
NODE_BLK = 1024
N_GRID = N_P // NODE_BLK
EDGE_BLK = 3584
E_GRID = 802816 // EDGE_BLK

PI = tuple((m // 32) * 32 + (m % 32) // 2 + (m % 2) * 16 for m in range(64))


def _leaky(v):
    return jnp.where(v >= 0, v, NEG_SLOPE * v)


def _node_mlp_body(x_ref, w1, b1, w2, b2, w3, b3, w3p, b3p,
                   out_ref, outp_ref):
    h = jnp.maximum(jnp.dot(x_ref[...], w1[...],
                            preferred_element_type=jnp.float32) + b1[...], 0.0)
    h = jnp.maximum(jnp.dot(h, w2[...],
                            preferred_element_type=jnp.float32) + b2[...], 0.0)
    out_ref[...] = jnp.dot(h, w3[...],
                           preferred_element_type=jnp.float32) + b3[...]
    outp_ref[...] = jnp.dot(h, w3p[...],
                            preferred_element_type=jnp.float32) + b3p[...]


def _node_mlp(x_p, p):
    full = lambda shape: pl.BlockSpec(shape, lambda i: (0,) * len(shape))
    return pl.pallas_call(
        _node_mlp_body,
        grid=(N_GRID,),
        in_specs=[
            pl.BlockSpec((NODE_BLK, DIN), lambda i: (i, 0)),
            full((DIN, H)), full((1, H)),
            full((H, H)), full((1, H)),
            full((H, HP)), full((1, HP)),
            full((H, H)), full((1, H)),
        ],
        out_specs=[pl.BlockSpec((NODE_BLK, HP), lambda i: (i, 0)),
                   pl.BlockSpec((NODE_BLK, H), lambda i: (i, 0))],
        out_shape=[jax.ShapeDtypeStruct((N_P, HP), jnp.float32),
                   jax.ShapeDtypeStruct((N_P, H), jnp.float32)],
    )(x_p, p['Wnx1'], p['bnx1'].reshape(1, H),
      p['Wnx2'], p['bnx2'].reshape(1, H),
      jnp.pad(p['Wnx3'], ((0, 0), (0, HP - H))),
      jnp.pad(p['bnx3'], (0, HP - H)).reshape(1, HP),
      p['Wnx3'][:, jnp.array(PI)], p['bnx3'][jnp.array(PI)].reshape(1, H))


def _edge_mlp_body(ea_ref, w1, b1, w2, b2, w3, b3, we, be, out_ref):
    t = jnp.maximum(jnp.dot(ea_ref[...], w1[...],
                            preferred_element_type=jnp.float32) + b1[...], 0.0)
    t = jnp.maximum(jnp.dot(t, w2[...],
                            preferred_element_type=jnp.float32) + b2[...], 0.0)
    w3e = jnp.dot(w3[...], we[...], preferred_element_type=jnp.float32)
    b3e = jnp.dot(b3[...], we[...], preferred_element_type=jnp.float32) + be[...]
    out_ref[...] = jnp.dot(t, w3e, preferred_element_type=jnp.float32) + b3e


def _edge_mlp(edge_attr, p):
    full = lambda shape: pl.BlockSpec(shape, lambda i: (0,) * len(shape))
    return pl.pallas_call(
        _edge_mlp_body,
        grid=(E_GRID,),
        in_specs=[
            pl.BlockSpec((EDGE_BLK, DE), lambda i: (i, 0)),
            full((DE, H)), full((1, H)),
            full((H, H)), full((1, H)),
            full((H, H)), full((1, H)),
            full((H, HP)), full((1, HP)),
        ],
        out_specs=pl.BlockSpec((EDGE_BLK, HP), lambda i: (i, 0)),
        out_shape=jax.ShapeDtypeStruct((E_P, HP), jnp.float32),
    )(edge_attr, p['Wne1'], p['bne1'].reshape(1, H),
      p['Wne2'], p['bne2'].reshape(1, H),
      p['Wne3'], p['bne3'].reshape(1, H),
      jnp.pad(p['We'], ((0, 0), (0, HP - H))),
      jnp.pad(p['be'], (0, HP - H)).reshape(1, HP))


def _head_body(h_ref, agg_ref, ids_ref, eps_ref, wg1, bg1, wg2, bg2,
               wo1, bo1, gamma, beta, wo2, bo2,
               o_ref, sig_ref, hp_ref):
    step = pl.program_id(0)

    @pl.when(step == 0)
    def _init():
        hp_ref[...] = jnp.full((G, H), -1e30, jnp.float32)

    z = ((1.0 + eps_ref[0, 0]) * h_ref[...]
         + agg_ref[...].astype(jnp.float32))
    z = _leaky(jnp.dot(z, wg1[...], preferred_element_type=jnp.float32)
               + bg1[...])
    z2 = jnp.dot(z, wg2[...], preferred_element_type=jnp.float32) + bg2[...]

    ids = ids_ref[...]
    gmin = jnp.min(ids)
    gmax = jnp.minimum(jnp.max(ids), G - 1)

    def body(g, _):
        mask = ids == g
        m = jnp.max(jnp.where(mask, z2, -1e30), axis=0, keepdims=True)
        cur = hp_ref[pl.ds(g, 1), :]
        hp_ref[pl.ds(g, 1), :] = jnp.maximum(cur, m)
        return 0

    jax.lax.fori_loop(gmin, gmax + 1, body, 0)

    @pl.when(step == N_GRID - 1)
    def _head():
        hp = hp_ref[...]
        hp = jnp.where(hp < -1e29, 0.0, hp)
        o = jnp.dot(hp, wo1[...], preferred_element_type=jnp.float32) + bo1[...]
        o = o * (1.0 / jnp.sqrt(1.0 + 1e-5)) * gamma[...] + beta[...]
        o = _leaky(o)
        o2 = jnp.dot(o, wo2[...], preferred_element_type=jnp.float32) + bo2[0, 0]
        o_ref[...] = o2
        sig_ref[...] = 1.0 / (1.0 + jnp.exp(-o2))


def _head(h_p, agg_p, batch2, p):
    full = lambda shape: pl.BlockSpec(shape, lambda i: (0,) * len(shape))
    return pl.pallas_call(
        _head_body,
        grid=(N_GRID,),
        in_specs=[
            pl.BlockSpec((NODE_BLK, H), lambda i: (i, 0)),
            pl.BlockSpec((NODE_BLK, H), lambda i: (i, 0)),
            pl.BlockSpec((NODE_BLK, 1), lambda i: (i, 0)),
            full((1, 1)),
            full((H, H)), full((1, H)),
            full((H, H)), full((1, H)),
            full((H, H)), full((1, H)),
            full((1, H)), full((1, H)),
            full((H, 1)), full((1, 1)),
        ],
        out_specs=[full((G, 1)), full((G, 1))],
        out_shape=[jax.ShapeDtypeStruct((G, 1), jnp.float32),
                   jax.ShapeDtypeStruct((G, 1), jnp.float32)],
        scratch_shapes=[pltpu.VMEM((G, H), jnp.float32)],
    )(h_p, agg_p, batch2, p['eps'].reshape(1, 1),
      p['Wg1'][jnp.array(PI), :], p['bg1'].reshape(1, H),
      p['Wg2'], p['bg2'].reshape(1, H),
      p['Wo1'], p['bo1'].reshape(1, H),
      p['gamma'].reshape(1, H), p['beta'].reshape(1, H),
      p['Wo2'], p['bo2'].reshape(1, 1))


NW = 32
R = N_P // NW
TRASH = R
SC_CHUNK = 4096
N_GROUPS = SC_CHUNK // 16
E_P = 802816
N_CHUNKS = E_P // SC_CHUNK
BATCH = 128
CAP = BATCH + SC_CHUNK + 16


def _sc_agg_body(h_hbm, eaw_hbm, src_hbm, dst_hbm, out_hbm,
                 dstv0, dstv1, srcv0, srcv1, svacc, pkacc, idbuf, dlbuf,
                 pbuf, hrows, erows, agg, sem1, sem2,
                 sd0, sd1, ss0, ss1):
    cid = lax.axis_index("c")
    sid = lax.axis_index("s")
    wid = sid * 2 + cid
    iota = lax.iota(jnp.int32, 16)
    zero16 = jnp.zeros((16,), jnp.float32)
    pbuf[pl.ds(0, 16)] = jnp.zeros((16,), jnp.int32)

    def apply_batch(lo):
        def unpack_body(g, _):
            v = pkacc[pl.ds(g * 16, 16)]
            idbuf[pl.ds(g * 16, 16)] = v & 0xFFFFF
            dlbuf[pl.ds(g * 16, 16)] = jax.lax.shift_right_logical(v, 20)
            return 0

        lax.fori_loop(0, BATCH // 16, unpack_body, 0)

        cp1 = pltpu.async_copy(h_hbm.at[svacc.at[pl.ds(0, BATCH)]],
                               hrows, sem1)
        cp2 = pltpu.async_copy(eaw_hbm.at[idbuf], erows, sem2)
        cp1.wait()
        cp2.wait()

        def edge_grp_body(g, _):
            dlv = dlbuf[pl.ds(g * 16, 16)] * 32
            for lane in range(16):
                i = g * 16 + lane
                rb = dlv[lane]
                for q in range(2):
                    hv0 = hrows[i, pl.ds(q * 32, 16)]
                    hv1 = hrows[i, pl.ds(q * 32 + 16, 16)]
                    ev0 = erows[i, pl.ds(q * 32, 16)]
                    ev1 = erows[i, pl.ds(q * 32 + 16, 16)]
                    m0 = jnp.maximum(hv0 + ev0, 0.0)
                    m1 = jnp.maximum(hv1 + ev1, 0.0)
                    mb0 = jax.lax.shift_right_logical(
                        plsc.bitcast(m0, jnp.int32) + 0x8000, 16)
                    mb1 = jax.lax.shift_right_logical(
                        plsc.bitcast(m1, jnp.int32) + 0x8000, 16)
                    cur = agg[pl.ds(rb + q * 16, 16)]
                    nlo = jnp.maximum(cur & 0xFFFF, mb0)
                    nhi = jnp.maximum(cur & -65536,
                                      jax.lax.shift_left(mb1, 16))
                    agg[pl.ds(rb + q * 16, 16)] = nlo | nhi
            return 0

        lax.fori_loop(0, BATCH // 16, edge_grp_body, 0)

    def shift_batch():
        def shift_body(j, _):
            for ref in (svacc, pkacc):
                ref[pl.ds(j * 16, 16)] = ref[pl.ds(BATCH + j * 16, 16)]
            return 0

        lax.fori_loop(0, SC_CHUNK // 16, shift_body, 0)

    if True:
        lo = wid * R
        hi = lo + R
        lo_vec = jnp.full((16,), lo, jnp.int32)
        hi_vec = jnp.full((16,), hi, jnp.int32)
        zero16i = jnp.zeros((16,), jnp.int32)

        def zero_body(r, _):
            for q in range(2):
                agg[pl.ds(r * 32 + q * 16, 16)] = zero16i
            return 0

        lax.fori_loop(0, R + 1, zero_body, 0, unroll=4)

        def start_stage(c, db, sb, semd, sems):
            base = c * SC_CHUNK
            pltpu.async_copy(dst_hbm.at[pl.ds(base, SC_CHUNK)], db, semd)
            pltpu.async_copy(src_hbm.at[pl.ds(base, SC_CHUNK)], sb, sems)

        start_stage(0, dstv0, srcv0, sd0, ss0)
        start_stage(1, dstv1, srcv1, sd1, ss1)

        def chunk_pair_body(cc, cnt):
            for b, (db, sb, semd, sems) in enumerate(
                    ((dstv0, srcv0, sd0, ss0), (dstv1, srcv1, sd1, ss1))):
                c = cc * 2 + b
                base = c * SC_CHUNK
                pltpu.make_async_copy(dst_hbm.at[pl.ds(0, SC_CHUNK)],
                                      db, semd).wait()
                pltpu.make_async_copy(src_hbm.at[pl.ds(0, SC_CHUNK)],
                                      sb, sems).wait()
                base_vec = jnp.full((16,), base, jnp.int32) + iota

                def count_body(g, qc):
                    d = db[pl.ds(g * 16, 16)]
                    m = (d >= lo_vec) & (d < hi_vec)
                    return qc + jnp.where(m, 1, 0).astype(jnp.int32)

                qc = lax.fori_loop(0, N_GROUPS, count_body,
                                   jnp.zeros((16,), jnp.int32), unroll=8)

                s = qc
                for sh in (1, 2, 4, 8):
                    pbuf[pl.ds(16, 16)] = s
                    s = s + plsc.load_gather(pbuf, [iota + (16 - sh)])
                excl = s - qc
                total = s[15]

                def fill_body(g, wp):
                    d = db[pl.ds(g * 16, 16)]
                    sv = sb[pl.ds(g * 16, 16)]
                    m = (d >= lo_vec) & (d < hi_vec)
                    dl = d - lo_vec
                    packed = (base_vec + g * 16) | jax.lax.shift_left(dl, 20)
                    dest = jnp.where(m, wp, CAP - 16 + iota)
                    plsc.store_scatter(svacc, [dest], sv)
                    plsc.store_scatter(pkacc, [dest], packed)
                    return wp + jnp.where(m, 1, 0).astype(jnp.int32)

                lax.fori_loop(0, N_GROUPS, fill_body,
                              jnp.full((16,), cnt, jnp.int32) + excl,
                              unroll=4)
                cnt = cnt + total

                @pl.when(c + 2 < N_CHUNKS)
                def _prefetch():
                    start_stage(c + 2, db, sb, semd, sems)

                def drain_cond(cc2):
                    return cc2 >= BATCH

                def drain_body(cc2):
                    apply_batch(lo)
                    shift_batch()
                    return cc2 - BATCH

                cnt = lax.while_loop(drain_cond, drain_body, cnt)
            return cnt

        cnt = lax.fori_loop(0, N_CHUNKS // 2, chunk_pair_body, jnp.int32(0))

        pad_pk = (jnp.full((16,), TRASH << 20, jnp.int32)
                  | (wid * SC_CHUNK + iota))
        for j in range(BATCH // 16):
            dest = jnp.full((16,), cnt, jnp.int32) + iota + j * 16
            plsc.store_scatter(svacc, [dest], lo_vec + iota)
            plsc.store_scatter(pkacc, [dest], pad_pk + j * 16)
        apply_batch(lo)

        pltpu.sync_copy(agg.at[pl.ds(0, R * 32)],
                        out_hbm.at[pl.ds(lo * 32, R * 32)])


def _sc_agg(h_p, eaw, src_p, dst_p):
    mesh = plsc.VectorSubcoreMesh(core_axis_name="c", subcore_axis_name="s")
    f = pl.kernel(
        _sc_agg_body,
        out_type=jax.ShapeDtypeStruct((N_P * 32,), jnp.int32),
        mesh=mesh,
        compiler_params=pltpu.CompilerParams(needs_layout_passes=False),
        scratch_types=[
            pltpu.VMEM((SC_CHUNK,), jnp.int32),
            pltpu.VMEM((SC_CHUNK,), jnp.int32),
            pltpu.VMEM((SC_CHUNK,), jnp.int32),
            pltpu.VMEM((SC_CHUNK,), jnp.int32),
            pltpu.VMEM((CAP,), jnp.int32),
            pltpu.VMEM((CAP,), jnp.int32),
            pltpu.VMEM((BATCH,), jnp.int32),
            pltpu.VMEM((BATCH,), jnp.int32),
            pltpu.VMEM((32,), jnp.int32),
            pltpu.VMEM((BATCH, HP), jnp.float32),
            pltpu.VMEM((BATCH, HP), jnp.float32),
            pltpu.VMEM(((R + 1) * 32,), jnp.int32),
            pltpu.SemaphoreType.DMA,
            pltpu.SemaphoreType.DMA,
            pltpu.SemaphoreType.DMA,
            pltpu.SemaphoreType.DMA,
            pltpu.SemaphoreType.DMA,
            pltpu.SemaphoreType.DMA,
        ],
    )
    return f(h_p, eaw, src_p, dst_p)


def kernel(x, edge_index, batch, edge_attr, params):
    p = params
    x_p = jnp.pad(x, ((0, N_P - N), (0, 0)))
    batch_p = jnp.pad(batch, (0, N_P - N), constant_values=G)
    batch2 = batch_p.reshape(N_P, 1)

    ea_p = jnp.pad(edge_attr, ((0, E_P - E), (0, 0)))
    src_p = jnp.pad(edge_index[0], (0, E_P - E))
    dst_p = jnp.pad(edge_index[1], (0, E_P - E), constant_values=N_P - 1)

    hb, h_perm = _node_mlp(x_p, p)
    eaw = _edge_mlp(ea_p, p)
    agg_i = _sc_agg(hb, eaw, src_p, dst_p)
    agg_p = jax.lax.bitcast_convert_type(
        agg_i, jnp.bfloat16).reshape(N_P, H)

    o, sig = _head(h_perm, agg_p, batch2, p)
    return (o, sig)


# speedup vs baseline: 2.2256x; 1.1659x over previous
import functools

import jax
import jax.numpy as jnp
from jax import lax
from jax.experimental import pallas as pl
from jax.experimental.pallas import tpu as pltpu
from jax.experimental.pallas import tpu_sc as plsc

N = 50000
E = 800000
DIN = 128
DE = 16
H = 64
G = 64
NEG_SLOPE = 0.01

N_P = 50176
HP = 128
NODE_BLK = 1024
N_GRID = N_P // NODE_BLK
EDGE_BLK = 3584
E_GRID = 802816 // EDGE_BLK

PI = tuple((m // 32) * 32 + (m % 32) // 2 + (m % 2) * 16 for m in range(64))


def _leaky(v):
    return jnp.where(v >= 0, v, NEG_SLOPE * v)


def _node_mlp_body(x_ref, w1, b1, w2, b2, w3, b3, w3p, b3p,
                   out_ref, outp_ref):
    h = jnp.maximum(jnp.dot(x_ref[...], w1[...],
                            preferred_element_type=jnp.float32) + b1[...], 0.0)
    h = jnp.maximum(jnp.dot(h, w2[...],
                            preferred_element_type=jnp.float32) + b2[...], 0.0)
    out_ref[...] = jnp.dot(h, w3[...],
                           preferred_element_type=jnp.float32) + b3[...]
    outp_ref[...] = jnp.dot(h, w3p[...],
                            preferred_element_type=jnp.float32) + b3p[...]


def _node_mlp(x_p, p):
    full = lambda shape: pl.BlockSpec(shape, lambda i: (0,) * len(shape))
    return pl.pallas_call(
        _node_mlp_body,
        grid=(N_GRID,),
        in_specs=[
            pl.BlockSpec((NODE_BLK, DIN), lambda i: (i, 0)),
            full((DIN, H)), full((1, H)),
            full((H, H)), full((1, H)),
            full((H, HP)), full((1, HP)),
            full((H, H)), full((1, H)),
        ],
        out_specs=[pl.BlockSpec((NODE_BLK, HP), lambda i: (i, 0)),
                   pl.BlockSpec((NODE_BLK, H), lambda i: (i, 0))],
        out_shape=[jax.ShapeDtypeStruct((N_P, HP), jnp.float32),
                   jax.ShapeDtypeStruct((N_P, H), jnp.float32)],
    )(x_p, p['Wnx1'], p['bnx1'].reshape(1, H),
      p['Wnx2'], p['bnx2'].reshape(1, H),
      jnp.pad(p['Wnx3'], ((0, 0), (0, HP - H))),
      jnp.pad(p['bnx3'], (0, HP - H)).reshape(1, HP),
      p['Wnx3'][:, jnp.array(PI)], p['bnx3'][jnp.array(PI)].reshape(1, H))


def _edge_mlp_body(ea_ref, w1, b1, w2, b2, w3, b3, we, be, out_ref):
    t = jnp.maximum(jnp.dot(ea_ref[...], w1[...],
                            preferred_element_type=jnp.float32) + b1[...], 0.0)
    t = jnp.maximum(jnp.dot(t, w2[...],
                            preferred_element_type=jnp.float32) + b2[...], 0.0)
    w3e = jnp.dot(w3[...], we[...], preferred_element_type=jnp.float32)
    b3e = jnp.dot(b3[...], we[...], preferred_element_type=jnp.float32) + be[...]
    out_ref[...] = jnp.dot(t, w3e, preferred_element_type=jnp.float32) + b3e


def _edge_mlp(edge_attr, p):
    full = lambda shape: pl.BlockSpec(shape, lambda i: (0,) * len(shape))
    return pl.pallas_call(
        _edge_mlp_body,
        grid=(E_GRID,),
        in_specs=[
            pl.BlockSpec((EDGE_BLK, DE), lambda i: (i, 0)),
            full((DE, H)), full((1, H)),
            full((H, H)), full((1, H)),
            full((H, H)), full((1, H)),
            full((H, HP)), full((1, HP)),
        ],
        out_specs=pl.BlockSpec((EDGE_BLK, HP), lambda i: (i, 0)),
        out_shape=jax.ShapeDtypeStruct((E_P, HP), jnp.float32),
    )(edge_attr, p['Wne1'], p['bne1'].reshape(1, H),
      p['Wne2'], p['bne2'].reshape(1, H),
      p['Wne3'], p['bne3'].reshape(1, H),
      jnp.pad(p['We'], ((0, 0), (0, HP - H))),
      jnp.pad(p['be'], (0, HP - H)).reshape(1, HP))


def _head_body(h_ref, agg_ref, ids_ref, eps_ref, wg1, bg1, wg2, bg2,
               wo1, bo1, gamma, beta, wo2, bo2,
               o_ref, sig_ref, hp_ref):
    step = pl.program_id(0)

    @pl.when(step == 0)
    def _init():
        hp_ref[...] = jnp.full((G, H), -1e30, jnp.float32)

    z = ((1.0 + eps_ref[0, 0]) * h_ref[...]
         + agg_ref[...].astype(jnp.float32))
    z = _leaky(jnp.dot(z, wg1[...], preferred_element_type=jnp.float32)
               + bg1[...])
    z2 = jnp.dot(z, wg2[...], preferred_element_type=jnp.float32) + bg2[...]

    ids = ids_ref[...]
    gmin = jnp.min(ids)
    gmax = jnp.minimum(jnp.max(ids), G - 1)

    def body(g, _):
        mask = ids == g
        m = jnp.max(jnp.where(mask, z2, -1e30), axis=0, keepdims=True)
        cur = hp_ref[pl.ds(g, 1), :]
        hp_ref[pl.ds(g, 1), :] = jnp.maximum(cur, m)
        return 0

    jax.lax.fori_loop(gmin, gmax + 1, body, 0)

    @pl.when(step == N_GRID - 1)
    def _head():
        hp = hp_ref[...]
        hp = jnp.where(hp < -1e29, 0.0, hp)
        o = jnp.dot(hp, wo1[...], preferred_element_type=jnp.float32) + bo1[...]
        o = o * (1.0 / jnp.sqrt(1.0 + 1e-5)) * gamma[...] + beta[...]
        o = _leaky(o)
        o2 = jnp.dot(o, wo2[...], preferred_element_type=jnp.float32) + bo2[0, 0]
        o_ref[...] = o2
        sig_ref[...] = 1.0 / (1.0 + jnp.exp(-o2))


def _head(h_p, agg_p, batch2, p):
    full = lambda shape: pl.BlockSpec(shape, lambda i: (0,) * len(shape))
    return pl.pallas_call(
        _head_body,
        grid=(N_GRID,),
        in_specs=[
            pl.BlockSpec((NODE_BLK, H), lambda i: (i, 0)),
            pl.BlockSpec((NODE_BLK, H), lambda i: (i, 0)),
            pl.BlockSpec((NODE_BLK, 1), lambda i: (i, 0)),
            full((1, 1)),
            full((H, H)), full((1, H)),
            full((H, H)), full((1, H)),
            full((H, H)), full((1, H)),
            full((1, H)), full((1, H)),
            full((H, 1)), full((1, 1)),
        ],
        out_specs=[full((G, 1)), full((G, 1))],
        out_shape=[jax.ShapeDtypeStruct((G, 1), jnp.float32),
                   jax.ShapeDtypeStruct((G, 1), jnp.float32)],
        scratch_shapes=[pltpu.VMEM((G, H), jnp.float32)],
    )(h_p, agg_p, batch2, p['eps'].reshape(1, 1),
      p['Wg1'][jnp.array(PI), :], p['bg1'].reshape(1, H),
      p['Wg2'], p['bg2'].reshape(1, H),
      p['Wo1'], p['bo1'].reshape(1, H),
      p['gamma'].reshape(1, H), p['beta'].reshape(1, H),
      p['Wo2'], p['bo2'].reshape(1, 1))


NW = 32
R = N_P // NW
TRASH = R
SC_CHUNK = 4096
N_GROUPS = SC_CHUNK // 16
E_P = 802816
N_CHUNKS = E_P // SC_CHUNK
BATCH = 128
CAP = BATCH + SC_CHUNK + 16


def _sc_agg_body(h_hbm, eaw_hbm, src_hbm, dst_hbm, out_hbm,
                 dstv0, dstv1, srcv0, srcv1, svacc, pkacc, idbuf, dlbuf,
                 gsv, pbuf, hrows, erows, agg, sem1, sem2,
                 sd0, sd1, ss0, ss1):
    cid = lax.axis_index("c")
    sid = lax.axis_index("s")
    wid = sid * 2 + cid
    iota = lax.iota(jnp.int32, 16)
    zero16 = jnp.zeros((16,), jnp.float32)
    pbuf[pl.ds(0, 16)] = jnp.zeros((16,), jnp.int32)

    def fire_batch():
        def unpack_body(g, _):
            v = pkacc[pl.ds(g * 16, 16)]
            idbuf[pl.ds(g * 16, 16)] = v & 0xFFFFF
            dlbuf[pl.ds(g * 16, 16)] = jax.lax.shift_right_logical(v, 20)
            gsv[pl.ds(g * 16, 16)] = svacc[pl.ds(g * 16, 16)]
            return 0

        lax.fori_loop(0, BATCH // 16, unpack_body, 0)
        pltpu.async_copy(h_hbm.at[gsv], hrows, sem1)
        pltpu.async_copy(eaw_hbm.at[idbuf], erows, sem2)

    def consume_batch():
        pltpu.make_async_copy(h_hbm.at[gsv], hrows, sem1).wait()
        pltpu.make_async_copy(eaw_hbm.at[idbuf], erows, sem2).wait()

        def edge_grp_body(g, _):
            dlv = dlbuf[pl.ds(g * 16, 16)] * 32
            for lane in range(16):
                i = g * 16 + lane
                rb = dlv[lane]
                for q in range(2):
                    hv0 = hrows[i, pl.ds(q * 32, 16)]
                    hv1 = hrows[i, pl.ds(q * 32 + 16, 16)]
                    ev0 = erows[i, pl.ds(q * 32, 16)]
                    ev1 = erows[i, pl.ds(q * 32 + 16, 16)]
                    m0 = jnp.maximum(hv0 + ev0, 0.0)
                    m1 = jnp.maximum(hv1 + ev1, 0.0)
                    mb0 = jax.lax.shift_right_logical(
                        plsc.bitcast(m0, jnp.int32) + 0x8000, 16)
                    mb1 = jax.lax.shift_right_logical(
                        plsc.bitcast(m1, jnp.int32) + 0x8000, 16)
                    cur = agg[pl.ds(rb + q * 16, 16)]
                    nlo = jnp.maximum(cur & 0xFFFF, mb0)
                    nhi = jnp.maximum(cur & -65536,
                                      jax.lax.shift_left(mb1, 16))
                    agg[pl.ds(rb + q * 16, 16)] = nlo | nhi
            return 0

        lax.fori_loop(0, BATCH // 16, edge_grp_body, 0)

    def shift_batch():
        def shift_body(j, _):
            for ref in (svacc, pkacc):
                ref[pl.ds(j * 16, 16)] = ref[pl.ds(BATCH + j * 16, 16)]
            return 0

        lax.fori_loop(0, SC_CHUNK // 16, shift_body, 0)

    if True:
        lo = wid * R
        hi = lo + R
        lo_vec = jnp.full((16,), lo, jnp.int32)
        hi_vec = jnp.full((16,), hi, jnp.int32)
        zero16i = jnp.zeros((16,), jnp.int32)

        def zero_body(r, _):
            for q in range(2):
                agg[pl.ds(r * 32 + q * 16, 16)] = zero16i
            return 0

        lax.fori_loop(0, R + 1, zero_body, 0, unroll=4)

        def start_stage(c, db, sb, semd, sems):
            base = c * SC_CHUNK
            pltpu.async_copy(dst_hbm.at[pl.ds(base, SC_CHUNK)], db, semd)
            pltpu.async_copy(src_hbm.at[pl.ds(base, SC_CHUNK)], sb, sems)

        start_stage(0, dstv0, srcv0, sd0, ss0)
        start_stage(1, dstv1, srcv1, sd1, ss1)

        def chunk_pair_body(cc, carry):
            cnt, inflight = carry
            for b, (db, sb, semd, sems) in enumerate(
                    ((dstv0, srcv0, sd0, ss0), (dstv1, srcv1, sd1, ss1))):
                c = cc * 2 + b
                base = c * SC_CHUNK
                pltpu.make_async_copy(dst_hbm.at[pl.ds(0, SC_CHUNK)],
                                      db, semd).wait()
                pltpu.make_async_copy(src_hbm.at[pl.ds(0, SC_CHUNK)],
                                      sb, sems).wait()
                base_vec = jnp.full((16,), base, jnp.int32) + iota

                def count_body(g, qc):
                    d = db[pl.ds(g * 16, 16)]
                    m = (d >= lo_vec) & (d < hi_vec)
                    return qc + jnp.where(m, 1, 0).astype(jnp.int32)

                qc = lax.fori_loop(0, N_GROUPS, count_body,
                                   jnp.zeros((16,), jnp.int32), unroll=8)

                s = qc
                for sh in (1, 2, 4, 8):
                    pbuf[pl.ds(16, 16)] = s
                    s = s + plsc.load_gather(pbuf, [iota + (16 - sh)])
                excl = s - qc
                total = s[15]

                def fill_body(g, wp):
                    d = db[pl.ds(g * 16, 16)]
                    sv = sb[pl.ds(g * 16, 16)]
                    m = (d >= lo_vec) & (d < hi_vec)
                    dl = d - lo_vec
                    packed = (base_vec + g * 16) | jax.lax.shift_left(dl, 20)
                    dest = jnp.where(m, wp, CAP - 16 + iota)
                    plsc.store_scatter(svacc, [dest], sv)
                    plsc.store_scatter(pkacc, [dest], packed)
                    return wp + jnp.where(m, 1, 0).astype(jnp.int32)

                lax.fori_loop(0, N_GROUPS, fill_body,
                              jnp.full((16,), cnt, jnp.int32) + excl,
                              unroll=4)
                cnt = cnt + total

                def drain_cond(carry):
                    return carry[0] >= BATCH

                def drain_body(carry):
                    cc2, infl = carry

                    @pl.when(infl == 1)
                    def _():
                        consume_batch()

                    fire_batch()
                    shift_batch()
                    return (cc2 - BATCH, jnp.int32(1))

                cnt, inflight = lax.while_loop(drain_cond, drain_body,
                                               (cnt, inflight))

                @pl.when(c + 2 < N_CHUNKS)
                def _prefetch():
                    start_stage(c + 2, db, sb, semd, sems)
            return (cnt, inflight)

        cnt, inflight = lax.fori_loop(0, N_CHUNKS // 2, chunk_pair_body,
                                      (jnp.int32(0), jnp.int32(0)))

        @pl.when(inflight == 1)
        def _final_consume():
            consume_batch()

        pad_pk = (jnp.full((16,), TRASH << 20, jnp.int32)
                  | (wid * SC_CHUNK + iota))
        for j in range(BATCH // 16):
            dest = jnp.full((16,), cnt, jnp.int32) + iota + j * 16
            plsc.store_scatter(svacc, [dest], lo_vec + iota)
            plsc.store_scatter(pkacc, [dest], pad_pk + j * 16)
        fire_batch()
        consume_batch()

        pltpu.sync_copy(agg.at[pl.ds(0, R * 32)],
                        out_hbm.at[pl.ds(lo * 32, R * 32)])


def _sc_agg(h_p, eaw, src_p, dst_p):
    mesh = plsc.VectorSubcoreMesh(core_axis_name="c", subcore_axis_name="s")
    f = pl.kernel(
        _sc_agg_body,
        out_type=jax.ShapeDtypeStruct((N_P * 32,), jnp.int32),
        mesh=mesh,
        compiler_params=pltpu.CompilerParams(needs_layout_passes=False),
        scratch_types=[
            pltpu.VMEM((SC_CHUNK,), jnp.int32),
            pltpu.VMEM((SC_CHUNK,), jnp.int32),
            pltpu.VMEM((SC_CHUNK,), jnp.int32),
            pltpu.VMEM((SC_CHUNK,), jnp.int32),
            pltpu.VMEM((CAP,), jnp.int32),
            pltpu.VMEM((CAP,), jnp.int32),
            pltpu.VMEM((BATCH,), jnp.int32),
            pltpu.VMEM((BATCH,), jnp.int32),
            pltpu.VMEM((BATCH,), jnp.int32),
            pltpu.VMEM((32,), jnp.int32),
            pltpu.VMEM((BATCH, HP), jnp.float32),
            pltpu.VMEM((BATCH, HP), jnp.float32),
            pltpu.VMEM(((R + 1) * 32,), jnp.int32),
            pltpu.SemaphoreType.DMA,
            pltpu.SemaphoreType.DMA,
            pltpu.SemaphoreType.DMA,
            pltpu.SemaphoreType.DMA,
            pltpu.SemaphoreType.DMA,
            pltpu.SemaphoreType.DMA,
        ],
    )
    return f(h_p, eaw, src_p, dst_p)


def kernel(x, edge_index, batch, edge_attr, params):
    p = params
    x_p = jnp.pad(x, ((0, N_P - N), (0, 0)))
    batch_p = jnp.pad(batch, (0, N_P - N), constant_values=G)
    batch2 = batch_p.reshape(N_P, 1)

    ea_p = jnp.pad(edge_attr, ((0, E_P - E), (0, 0)))
    src_p = jnp.pad(edge_index[0], (0, E_P - E))
    dst_p = jnp.pad(edge_index[1], (0, E_P - E), constant_values=N_P - 1)

    hb, h_perm = _node_mlp(x_p, p)
    eaw = _edge_mlp(ea_p, p)
    agg_i = _sc_agg(hb, eaw, src_p, dst_p)
    agg_p = jax.lax.bitcast_convert_type(
        agg_i, jnp.bfloat16).reshape(N_P, H)

    o, sig = _head(h_perm, agg_p, batch2, p)
    return (o, sig)


# speedup vs baseline: 2.2355x; 1.0044x over previous
import functools

import jax
import jax.numpy as jnp
from jax import lax
from jax.experimental import pallas as pl
from jax.experimental.pallas import tpu as pltpu
from jax.experimental.pallas import tpu_sc as plsc

N = 50000
E = 800000
DIN = 128
DE = 16
H = 64
G = 64
NEG_SLOPE = 0.01

N_P = 50176
HP = 128
NODE_BLK = 1024
N_GRID = N_P // NODE_BLK
EDGE_BLK = 3584
E_GRID = 802816 // EDGE_BLK

PI = tuple((m // 32) * 32 + (m % 32) // 2 + (m % 2) * 16 for m in range(64))


def _leaky(v):
    return jnp.where(v >= 0, v, NEG_SLOPE * v)


def _node_mlp_body(x_ref, w1, b1, w2, b2, w3, b3, out_ref, outp_ref):
    h = jnp.maximum(jnp.dot(x_ref[...], w1[...],
                            preferred_element_type=jnp.float32) + b1[...], 0.0)
    h = jnp.maximum(jnp.dot(h, w2[...],
                            preferred_element_type=jnp.float32) + b2[...], 0.0)
    hp = jnp.dot(h, w3[...], preferred_element_type=jnp.float32) + b3[...]
    out_ref[...] = hp
    outp_ref[...] = hp[:, :H]


def _node_mlp(x_p, p):
    full = lambda shape: pl.BlockSpec(shape, lambda i: (0,) * len(shape))
    return pl.pallas_call(
        _node_mlp_body,
        grid=(N_GRID,),
        in_specs=[
            pl.BlockSpec((NODE_BLK, DIN), lambda i: (i, 0)),
            full((DIN, H)), full((1, H)),
            full((H, H)), full((1, H)),
            full((H, HP)), full((1, HP)),
        ],
        out_specs=[pl.BlockSpec((NODE_BLK, HP), lambda i: (i, 0)),
                   pl.BlockSpec((NODE_BLK, H), lambda i: (i, 0))],
        out_shape=[jax.ShapeDtypeStruct((N_P, HP), jnp.float32),
                   jax.ShapeDtypeStruct((N_P, H), jnp.float32)],
    )(x_p, p['Wnx1'], p['bnx1'].reshape(1, H),
      p['Wnx2'], p['bnx2'].reshape(1, H),
      jnp.pad(p['Wnx3'], ((0, 0), (0, HP - H))),
      jnp.pad(p['bnx3'], (0, HP - H)).reshape(1, HP))


def _edge_mlp_body(ea_ref, w1, b1, w2, b2, w3, b3, we, be, out_ref):
    t = jnp.maximum(jnp.dot(ea_ref[...], w1[...],
                            preferred_element_type=jnp.float32) + b1[...], 0.0)
    t = jnp.maximum(jnp.dot(t, w2[...],
                            preferred_element_type=jnp.float32) + b2[...], 0.0)
    w3e = jnp.dot(w3[...], we[...], preferred_element_type=jnp.float32)
    b3e = jnp.dot(b3[...], we[...], preferred_element_type=jnp.float32) + be[...]
    out_ref[...] = jnp.dot(t, w3e, preferred_element_type=jnp.float32) + b3e


def _edge_mlp(edge_attr, p):
    full = lambda shape: pl.BlockSpec(shape, lambda i: (0,) * len(shape))
    return pl.pallas_call(
        _edge_mlp_body,
        grid=(E_GRID,),
        in_specs=[
            pl.BlockSpec((EDGE_BLK, DE), lambda i: (i, 0)),
            full((DE, H)), full((1, H)),
            full((H, H)), full((1, H)),
            full((H, H)), full((1, H)),
            full((H, HP)), full((1, HP)),
        ],
        out_specs=pl.BlockSpec((EDGE_BLK, HP), lambda i: (i, 0)),
        out_shape=jax.ShapeDtypeStruct((E_P, HP), jnp.float32),
    )(edge_attr, p['Wne1'], p['bne1'].reshape(1, H),
      p['Wne2'], p['bne2'].reshape(1, H),
      p['Wne3'], p['bne3'].reshape(1, H),
      jnp.pad(p['We'], ((0, 0), (0, HP - H))),
      jnp.pad(p['be'], (0, HP - H)).reshape(1, HP))


def _head_body(h_ref, agg_ref, ids_ref, eps_ref, wg1, bg1, wg2, bg2,
               wo1, bo1, gamma, beta, wo2, bo2,
               o_ref, sig_ref, hp_ref):
    step = pl.program_id(0)

    @pl.when(step == 0)
    def _init():
        hp_ref[...] = jnp.full((G, H), -1e30, jnp.float32)

    z = (1.0 + eps_ref[0, 0]) * h_ref[...] + agg_ref[:, :H]
    z = _leaky(jnp.dot(z, wg1[...], preferred_element_type=jnp.float32)
               + bg1[...])
    z2 = jnp.dot(z, wg2[...], preferred_element_type=jnp.float32) + bg2[...]

    ids = ids_ref[...]
    gmin = jnp.min(ids)
    gmax = jnp.minimum(jnp.max(ids), G - 1)

    def body(g, _):
        mask = ids == g
        m = jnp.max(jnp.where(mask, z2, -1e30), axis=0, keepdims=True)
        cur = hp_ref[pl.ds(g, 1), :]
        hp_ref[pl.ds(g, 1), :] = jnp.maximum(cur, m)
        return 0

    jax.lax.fori_loop(gmin, gmax + 1, body, 0)

    @pl.when(step == N_GRID - 1)
    def _head():
        hp = hp_ref[...]
        hp = jnp.where(hp < -1e29, 0.0, hp)
        o = jnp.dot(hp, wo1[...], preferred_element_type=jnp.float32) + bo1[...]
        o = o * (1.0 / jnp.sqrt(1.0 + 1e-5)) * gamma[...] + beta[...]
        o = _leaky(o)
        o2 = jnp.dot(o, wo2[...], preferred_element_type=jnp.float32) + bo2[0, 0]
        o_ref[...] = o2
        sig_ref[...] = 1.0 / (1.0 + jnp.exp(-o2))


def _head(h_p, agg_p, batch2, p):
    full = lambda shape: pl.BlockSpec(shape, lambda i: (0,) * len(shape))
    return pl.pallas_call(
        _head_body,
        grid=(N_GRID,),
        in_specs=[
            pl.BlockSpec((NODE_BLK, H), lambda i: (i, 0)),
            pl.BlockSpec((NODE_BLK, HP), lambda i: (i, 0)),
            pl.BlockSpec((NODE_BLK, 1), lambda i: (i, 0)),
            full((1, 1)),
            full((H, H)), full((1, H)),
            full((H, H)), full((1, H)),
            full((H, H)), full((1, H)),
            full((1, H)), full((1, H)),
            full((H, 1)), full((1, 1)),
        ],
        out_specs=[full((G, 1)), full((G, 1))],
        out_shape=[jax.ShapeDtypeStruct((G, 1), jnp.float32),
                   jax.ShapeDtypeStruct((G, 1), jnp.float32)],
        scratch_shapes=[pltpu.VMEM((G, H), jnp.float32)],
    )(h_p, agg_p, batch2, p['eps'].reshape(1, 1),
      p['Wg1'], p['bg1'].reshape(1, H),
      p['Wg2'], p['bg2'].reshape(1, H),
      p['Wo1'], p['bo1'].reshape(1, H),
      p['gamma'].reshape(1, H), p['beta'].reshape(1, H),
      p['Wo2'], p['bo2'].reshape(1, 1))


NW = 32
R = N_P // NW
TRASH = R
SC_CHUNK = 4096
N_GROUPS = SC_CHUNK // 16
E_P = 802816
N_CHUNKS = E_P // SC_CHUNK
BATCH = 128
CAP = BATCH + SC_CHUNK + 16


def _sc_agg_body(h_hbm, eaw_hbm, src_hbm, dst_hbm, out_hbm,
                 dstv0, dstv1, srcv0, srcv1, svacc, pkacc, idbuf, dlbuf,
                 gsv, stg, pbuf, hrows, erows, agg, sem1, sem2,
                 sd0, sd1, ss0, ss1):
    cid = lax.axis_index("c")
    sid = lax.axis_index("s")
    wid = sid * 2 + cid
    iota = lax.iota(jnp.int32, 16)
    zero16 = jnp.zeros((16,), jnp.float32)
    pbuf[pl.ds(0, 16)] = jnp.zeros((16,), jnp.int32)

    def fire_batch():
        def unpack_body(g, _):
            v = pkacc[pl.ds(g * 16, 16)]
            idbuf[pl.ds(g * 16, 16)] = v & 0xFFFFF
            dlbuf[pl.ds(g * 16, 16)] = jax.lax.shift_right_logical(v, 20)
            gsv[pl.ds(g * 16, 16)] = svacc[pl.ds(g * 16, 16)]
            return 0

        lax.fori_loop(0, BATCH // 16, unpack_body, 0)
        pltpu.async_copy(h_hbm.at[gsv], hrows, sem1)
        pltpu.async_copy(eaw_hbm.at[idbuf], erows, sem2)

    def consume_batch():
        pltpu.make_async_copy(h_hbm.at[gsv], hrows, sem1).wait()
        pltpu.make_async_copy(eaw_hbm.at[idbuf], erows, sem2).wait()

        def edge_grp_body(g, _):
            dlv = dlbuf[pl.ds(g * 16, 16)] * 32
            for lane in range(16):
                i = g * 16 + lane
                rb = dlv[lane]
                for q in range(2):
                    hv0 = hrows[i, pl.ds(q * 32, 16)]
                    hv1 = hrows[i, pl.ds(q * 32 + 16, 16)]
                    ev0 = erows[i, pl.ds(q * 32, 16)]
                    ev1 = erows[i, pl.ds(q * 32 + 16, 16)]
                    m0 = jnp.maximum(hv0 + ev0, 0.0)
                    m1 = jnp.maximum(hv1 + ev1, 0.0)
                    mb0 = jax.lax.shift_right_logical(
                        plsc.bitcast(m0, jnp.int32) + 0x8000, 16)
                    mb1 = jax.lax.shift_right_logical(
                        plsc.bitcast(m1, jnp.int32) + 0x8000, 16)
                    cur = agg[pl.ds(rb + q * 16, 16)]
                    nlo = jnp.maximum(cur & 0xFFFF, mb0)
                    nhi = jnp.maximum(cur & -65536,
                                      jax.lax.shift_left(mb1, 16))
                    agg[pl.ds(rb + q * 16, 16)] = nlo | nhi
            return 0

        lax.fori_loop(0, BATCH // 16, edge_grp_body, 0)

    def shift_batch():
        def shift_body(j, _):
            for ref in (svacc, pkacc):
                ref[pl.ds(j * 16, 16)] = ref[pl.ds(BATCH + j * 16, 16)]
            return 0

        lax.fori_loop(0, SC_CHUNK // 16, shift_body, 0)

    if True:
        lo = wid * R
        hi = lo + R
        lo_vec = jnp.full((16,), lo, jnp.int32)
        hi_vec = jnp.full((16,), hi, jnp.int32)
        zero16i = jnp.zeros((16,), jnp.int32)

        def zero_body(r, _):
            for q in range(2):
                agg[pl.ds(r * 32 + q * 16, 16)] = zero16i
            return 0

        lax.fori_loop(0, R + 1, zero_body, 0, unroll=4)

        def start_stage(c, db, sb, semd, sems):
            base = c * SC_CHUNK
            pltpu.async_copy(dst_hbm.at[pl.ds(base, SC_CHUNK)], db, semd)
            pltpu.async_copy(src_hbm.at[pl.ds(base, SC_CHUNK)], sb, sems)

        start_stage(0, dstv0, srcv0, sd0, ss0)
        start_stage(1, dstv1, srcv1, sd1, ss1)

        def chunk_pair_body(cc, carry):
            cnt, inflight = carry
            for b, (db, sb, semd, sems) in enumerate(
                    ((dstv0, srcv0, sd0, ss0), (dstv1, srcv1, sd1, ss1))):
                c = cc * 2 + b
                base = c * SC_CHUNK
                pltpu.make_async_copy(dst_hbm.at[pl.ds(0, SC_CHUNK)],
                                      db, semd).wait()
                pltpu.make_async_copy(src_hbm.at[pl.ds(0, SC_CHUNK)],
                                      sb, sems).wait()
                base_vec = jnp.full((16,), base, jnp.int32) + iota

                def count_body(g, qc):
                    d = db[pl.ds(g * 16, 16)]
                    m = (d >= lo_vec) & (d < hi_vec)
                    return qc + jnp.where(m, 1, 0).astype(jnp.int32)

                qc = lax.fori_loop(0, N_GROUPS, count_body,
                                   jnp.zeros((16,), jnp.int32), unroll=16)

                s = qc
                for sh in (1, 2, 4, 8):
                    pbuf[pl.ds(16, 16)] = s
                    s = s + plsc.load_gather(pbuf, [iota + (16 - sh)])
                excl = s - qc
                total = s[15]

                def fill_body(g, wp):
                    d = db[pl.ds(g * 16, 16)]
                    sv = sb[pl.ds(g * 16, 16)]
                    m = (d >= lo_vec) & (d < hi_vec)
                    dl = d - lo_vec
                    packed = (base_vec + g * 16) | jax.lax.shift_left(dl, 20)
                    dest = jnp.where(m, wp, CAP - 16 + iota)
                    plsc.store_scatter(svacc, [dest], sv)
                    plsc.store_scatter(pkacc, [dest], packed)
                    return wp + jnp.where(m, 1, 0).astype(jnp.int32)

                lax.fori_loop(0, N_GROUPS, fill_body,
                              jnp.full((16,), cnt, jnp.int32) + excl,
                              unroll=8)
                cnt = cnt + total

                def drain_cond(carry):
                    return carry[0] >= BATCH

                def drain_body(carry):
                    cc2, infl = carry

                    @pl.when(infl == 1)
                    def _():
                        consume_batch()

                    fire_batch()
                    shift_batch()
                    return (cc2 - BATCH, jnp.int32(1))

                cnt, inflight = lax.while_loop(drain_cond, drain_body,
                                               (cnt, inflight))

                @pl.when(c + 2 < N_CHUNKS)
                def _prefetch():
                    start_stage(c + 2, db, sb, semd, sems)
            return (cnt, inflight)

        cnt, inflight = lax.fori_loop(0, N_CHUNKS // 2, chunk_pair_body,
                                      (jnp.int32(0), jnp.int32(0)))

        @pl.when(inflight == 1)
        def _final_consume():
            consume_batch()

        pad_pk = (jnp.full((16,), TRASH << 20, jnp.int32)
                  | (wid * SC_CHUNK + iota))
        for j in range(BATCH // 16):
            dest = jnp.full((16,), cnt, jnp.int32) + iota + j * 16
            plsc.store_scatter(svacc, [dest], lo_vec + iota)
            plsc.store_scatter(pkacc, [dest], pad_pk + j * 16)
        fire_batch()
        consume_batch()

        def out_body(ob, _):
            for rr in range(32):
                r = ob * 32 + rr
                w0 = agg[pl.ds(r * 32, 16)]
                w1 = agg[pl.ds(r * 32 + 16, 16)]
                stg[rr, pl.ds(0, 16)] = plsc.bitcast(
                    jax.lax.shift_left(w0, 16), jnp.float32)
                stg[rr, pl.ds(16, 16)] = plsc.bitcast(w0 & -65536,
                                                      jnp.float32)
                stg[rr, pl.ds(32, 16)] = plsc.bitcast(
                    jax.lax.shift_left(w1, 16), jnp.float32)
                stg[rr, pl.ds(48, 16)] = plsc.bitcast(w1 & -65536,
                                                      jnp.float32)
            pltpu.sync_copy(stg, out_hbm.at[pl.ds(lo + ob * 32, 32), :])
            return 0

        lax.fori_loop(0, R // 32, out_body, 0)


def _sc_agg(h_p, eaw, src_p, dst_p):
    mesh = plsc.VectorSubcoreMesh(core_axis_name="c", subcore_axis_name="s")
    f = pl.kernel(
        _sc_agg_body,
        out_type=jax.ShapeDtypeStruct((N_P, HP), jnp.float32),
        mesh=mesh,
        compiler_params=pltpu.CompilerParams(needs_layout_passes=False),
        scratch_types=[
            pltpu.VMEM((SC_CHUNK,), jnp.int32),
            pltpu.VMEM((SC_CHUNK,), jnp.int32),
            pltpu.VMEM((SC_CHUNK,), jnp.int32),
            pltpu.VMEM((SC_CHUNK,), jnp.int32),
            pltpu.VMEM((CAP,), jnp.int32),
            pltpu.VMEM((CAP,), jnp.int32),
            pltpu.VMEM((BATCH,), jnp.int32),
            pltpu.VMEM((BATCH,), jnp.int32),
            pltpu.VMEM((BATCH,), jnp.int32),
            pltpu.VMEM((32, HP), jnp.float32),
            pltpu.VMEM((32,), jnp.int32),
            pltpu.VMEM((BATCH, HP), jnp.float32),
            pltpu.VMEM((BATCH, HP), jnp.float32),
            pltpu.VMEM(((R + 1) * 32,), jnp.int32),
            pltpu.SemaphoreType.DMA,
            pltpu.SemaphoreType.DMA,
            pltpu.SemaphoreType.DMA,
            pltpu.SemaphoreType.DMA,
            pltpu.SemaphoreType.DMA,
            pltpu.SemaphoreType.DMA,
        ],
    )
    return f(h_p, eaw, src_p, dst_p)


def kernel(x, edge_index, batch, edge_attr, params):
    p = params
    x_p = jnp.pad(x, ((0, N_P - N), (0, 0)))
    batch_p = jnp.pad(batch, (0, N_P - N), constant_values=G)
    batch2 = batch_p.reshape(N_P, 1)

    ea_p = jnp.pad(edge_attr, ((0, E_P - E), (0, 0)))
    src_p = jnp.pad(edge_index[0], (0, E_P - E))
    dst_p = jnp.pad(edge_index[1], (0, E_P - E), constant_values=N_P - 1)

    hb, h_f = _node_mlp(x_p, p)
    eaw = _edge_mlp(ea_p, p)
    agg_p = _sc_agg(hb, eaw, src_p, dst_p)

    o, sig = _head(h_f, agg_p, batch2, p)
    return (o, sig)


# speedup vs baseline: 2.5542x; 1.1426x over previous
import functools

import jax
import jax.numpy as jnp
from jax import lax
from jax.experimental import pallas as pl
from jax.experimental.pallas import tpu as pltpu
from jax.experimental.pallas import tpu_sc as plsc

N = 50000
E = 800000
DIN = 128
DE = 16
H = 64
G = 64
NEG_SLOPE = 0.01

N_P = 50176
HP = 128
NODE_BLK = 1024
N_GRID = N_P // NODE_BLK
EDGE_BLK = 3584
E_GRID = 802816 // EDGE_BLK

PI = tuple((m // 32) * 32 + (m % 32) // 2 + (m % 2) * 16 for m in range(64))


def _leaky(v):
    return jnp.where(v >= 0, v, NEG_SLOPE * v)


def _node_mlp_body(x_ref, w1, b1, w2, b2, w3, b3, out_ref, outp_ref):
    h = jnp.maximum(jnp.dot(x_ref[...], w1[...],
                            preferred_element_type=jnp.float32) + b1[...], 0.0)
    h = jnp.maximum(jnp.dot(h, w2[...],
                            preferred_element_type=jnp.float32) + b2[...], 0.0)
    hp = jnp.dot(h, w3[...], preferred_element_type=jnp.float32) + b3[...]
    out_ref[...] = hp
    outp_ref[...] = hp[:, :H]


def _node_mlp(x_p, p):
    full = lambda shape: pl.BlockSpec(shape, lambda i: (0,) * len(shape))
    return pl.pallas_call(
        _node_mlp_body,
        grid=(N_GRID,),
        in_specs=[
            pl.BlockSpec((NODE_BLK, DIN), lambda i: (i, 0)),
            full((DIN, H)), full((1, H)),
            full((H, H)), full((1, H)),
            full((H, HP)), full((1, HP)),
        ],
        out_specs=[pl.BlockSpec((NODE_BLK, HP), lambda i: (i, 0)),
                   pl.BlockSpec((NODE_BLK, H), lambda i: (i, 0))],
        out_shape=[jax.ShapeDtypeStruct((N_P, HP), jnp.float32),
                   jax.ShapeDtypeStruct((N_P, H), jnp.float32)],
    )(x_p, p['Wnx1'], p['bnx1'].reshape(1, H),
      p['Wnx2'], p['bnx2'].reshape(1, H),
      jnp.pad(p['Wnx3'], ((0, 0), (0, HP - H))),
      jnp.pad(p['bnx3'], (0, HP - H)).reshape(1, HP))


def _edge_mlp_body(ea_ref, w1, b1, w2, b2, w3, b3, we, be, out_ref):
    t = jnp.maximum(jnp.dot(ea_ref[...], w1[...],
                            preferred_element_type=jnp.float32) + b1[...], 0.0)
    t = jnp.maximum(jnp.dot(t, w2[...],
                            preferred_element_type=jnp.float32) + b2[...], 0.0)
    w3e = jnp.dot(w3[...], we[...], preferred_element_type=jnp.float32)
    b3e = jnp.dot(b3[...], we[...], preferred_element_type=jnp.float32) + be[...]
    out_ref[...] = jnp.dot(t, w3e, preferred_element_type=jnp.float32) + b3e


def _edge_mlp(edge_attr, p):
    full = lambda shape: pl.BlockSpec(shape, lambda i: (0,) * len(shape))
    return pl.pallas_call(
        _edge_mlp_body,
        grid=(E_GRID,),
        in_specs=[
            pl.BlockSpec((EDGE_BLK, DE), lambda i: (i, 0)),
            full((DE, H)), full((1, H)),
            full((H, H)), full((1, H)),
            full((H, H)), full((1, H)),
            full((H, HP)), full((1, HP)),
        ],
        out_specs=pl.BlockSpec((EDGE_BLK, HP), lambda i: (i, 0)),
        out_shape=jax.ShapeDtypeStruct((E_P, HP), jnp.float32),
    )(edge_attr, p['Wne1'], p['bne1'].reshape(1, H),
      p['Wne2'], p['bne2'].reshape(1, H),
      p['Wne3'], p['bne3'].reshape(1, H),
      jnp.pad(p['We'], ((0, 0), (0, HP - H))),
      jnp.pad(p['be'], (0, HP - H)).reshape(1, HP))


def _head_body(h_ref, agg_ref, ids_ref, eps_ref, wg1, bg1, wg2, bg2,
               wo1, bo1, gamma, beta, wo2, bo2,
               o_ref, sig_ref, hp_ref):
    step = pl.program_id(0)

    @pl.when(step == 0)
    def _init():
        hp_ref[...] = jnp.full((G, H), -1e30, jnp.float32)

    z = (1.0 + eps_ref[0, 0]) * h_ref[...] + agg_ref[:, :H]
    z = _leaky(jnp.dot(z, wg1[...], preferred_element_type=jnp.float32)
               + bg1[...])
    z2 = jnp.dot(z, wg2[...], preferred_element_type=jnp.float32) + bg2[...]

    ids = ids_ref[...]
    gmin = jnp.min(ids)
    gmax = jnp.minimum(jnp.max(ids), G - 1)

    def body(g, _):
        mask = ids == g
        m = jnp.max(jnp.where(mask, z2, -1e30), axis=0, keepdims=True)
        cur = hp_ref[pl.ds(g, 1), :]
        hp_ref[pl.ds(g, 1), :] = jnp.maximum(cur, m)
        return 0

    jax.lax.fori_loop(gmin, gmax + 1, body, 0)

    @pl.when(step == N_GRID - 1)
    def _head():
        hp = hp_ref[...]
        hp = jnp.where(hp < -1e29, 0.0, hp)
        o = jnp.dot(hp, wo1[...], preferred_element_type=jnp.float32) + bo1[...]
        o = o * (1.0 / jnp.sqrt(1.0 + 1e-5)) * gamma[...] + beta[...]
        o = _leaky(o)
        o2 = jnp.dot(o, wo2[...], preferred_element_type=jnp.float32) + bo2[0, 0]
        o_ref[...] = o2
        sig_ref[...] = 1.0 / (1.0 + jnp.exp(-o2))


def _head(h_p, agg_p, batch2, p):
    full = lambda shape: pl.BlockSpec(shape, lambda i: (0,) * len(shape))
    return pl.pallas_call(
        _head_body,
        grid=(N_GRID,),
        in_specs=[
            pl.BlockSpec((NODE_BLK, H), lambda i: (i, 0)),
            pl.BlockSpec((NODE_BLK, HP), lambda i: (i, 0)),
            pl.BlockSpec((NODE_BLK, 1), lambda i: (i, 0)),
            full((1, 1)),
            full((H, H)), full((1, H)),
            full((H, H)), full((1, H)),
            full((H, H)), full((1, H)),
            full((1, H)), full((1, H)),
            full((H, 1)), full((1, 1)),
        ],
        out_specs=[full((G, 1)), full((G, 1))],
        out_shape=[jax.ShapeDtypeStruct((G, 1), jnp.float32),
                   jax.ShapeDtypeStruct((G, 1), jnp.float32)],
        scratch_shapes=[pltpu.VMEM((G, H), jnp.float32)],
    )(h_p, agg_p, batch2, p['eps'].reshape(1, 1),
      p['Wg1'], p['bg1'].reshape(1, H),
      p['Wg2'], p['bg2'].reshape(1, H),
      p['Wo1'], p['bo1'].reshape(1, H),
      p['gamma'].reshape(1, H), p['beta'].reshape(1, H),
      p['Wo2'], p['bo2'].reshape(1, 1))


NW = 32
R = N_P // NW
TRASH = R
SC_CHUNK = 4096
N_GROUPS = SC_CHUNK // 16
E_P = 802816
N_CHUNKS = E_P // SC_CHUNK
BATCH = 128
CAP = BATCH + SC_CHUNK + 16


def _sc_agg_body(h_hbm, eaw_hbm, src_hbm, dst_hbm, out_hbm,
                 dstv0, dstv1, srcv0, srcv1, svacc, pkacc, idbuf, dlbuf,
                 gsv, stg, pbuf, hrows, erows, agg, sem1, sem2,
                 sd0, sd1, ss0, ss1):
    cid = lax.axis_index("c")
    sid = lax.axis_index("s")
    wid = sid * 2 + cid
    iota = lax.iota(jnp.int32, 16)
    zero16 = jnp.zeros((16,), jnp.float32)
    pbuf[pl.ds(0, 16)] = jnp.zeros((16,), jnp.int32)

    def fire_batch():
        def unpack_body(g, _):
            v = pkacc[pl.ds(g * 16, 16)]
            idbuf[pl.ds(g * 16, 16)] = v & 0xFFFFF
            dlbuf[pl.ds(g * 16, 16)] = jax.lax.shift_right_logical(v, 20)
            gsv[pl.ds(g * 16, 16)] = svacc[pl.ds(g * 16, 16)]
            return 0

        lax.fori_loop(0, BATCH // 16, unpack_body, 0)
        pltpu.async_copy(h_hbm.at[gsv], hrows, sem1)
        pltpu.async_copy(eaw_hbm.at[idbuf], erows, sem2)

    def consume_batch():
        pltpu.make_async_copy(h_hbm.at[gsv], hrows, sem1).wait()
        pltpu.make_async_copy(eaw_hbm.at[idbuf], erows, sem2).wait()

        def edge_grp_body(g, _):
            dlv = dlbuf[pl.ds(g * 16, 16)] * 32
            for lane in range(16):
                i = g * 16 + lane
                rb = dlv[lane]
                for q in range(2):
                    hv0 = hrows[i, pl.ds(q * 32, 16)]
                    hv1 = hrows[i, pl.ds(q * 32 + 16, 16)]
                    ev0 = erows[i, pl.ds(q * 32, 16)]
                    ev1 = erows[i, pl.ds(q * 32 + 16, 16)]
                    m0 = jnp.maximum(hv0 + ev0, 0.0)
                    m1 = jnp.maximum(hv1 + ev1, 0.0)
                    mb0 = jax.lax.shift_right_logical(
                        plsc.bitcast(m0, jnp.int32) + 0x8000, 16)
                    mb1 = jax.lax.shift_right_logical(
                        plsc.bitcast(m1, jnp.int32) + 0x8000, 16)
                    cur = agg[pl.ds(rb + q * 16, 16)]
                    nlo = jnp.maximum(cur & 0xFFFF, mb0)
                    nhi = jnp.maximum(cur & -65536,
                                      jax.lax.shift_left(mb1, 16))
                    agg[pl.ds(rb + q * 16, 16)] = nlo | nhi
            return 0

        lax.fori_loop(0, BATCH // 16, edge_grp_body, 0)

    def shift_batch():
        def shift_body(j, _):
            for ref in (svacc, pkacc):
                ref[pl.ds(j * 16, 16)] = ref[pl.ds(BATCH + j * 16, 16)]
            return 0

        lax.fori_loop(0, SC_CHUNK // 16, shift_body, 0)

    if True:
        lo = wid * R
        hi = lo + R
        lo_vec = jnp.full((16,), lo, jnp.int32)
        hi_vec = jnp.full((16,), hi, jnp.int32)
        zero16i = jnp.zeros((16,), jnp.int32)

        def zero_body(r, _):
            for q in range(2):
                agg[pl.ds(r * 32 + q * 16, 16)] = zero16i
            return 0

        lax.fori_loop(0, R + 1, zero_body, 0, unroll=4)

        def start_stage(c, db, sb, semd, sems):
            base = c * SC_CHUNK
            pltpu.async_copy(dst_hbm.at[pl.ds(base, SC_CHUNK)], db, semd)
            pltpu.async_copy(src_hbm.at[pl.ds(base, SC_CHUNK)], sb, sems)

        start_stage(0, dstv0, srcv0, sd0, ss0)
        start_stage(1, dstv1, srcv1, sd1, ss1)

        def chunk_pair_body(cc, carry):
            cnt, inflight = carry
            for b, (db, sb, semd, sems) in enumerate(
                    ((dstv0, srcv0, sd0, ss0), (dstv1, srcv1, sd1, ss1))):
                c = cc * 2 + b
                base = c * SC_CHUNK
                pltpu.make_async_copy(dst_hbm.at[pl.ds(0, SC_CHUNK)],
                                      db, semd).wait()
                pltpu.make_async_copy(src_hbm.at[pl.ds(0, SC_CHUNK)],
                                      sb, sems).wait()
                base_vec = jnp.full((16,), base, jnp.int32) + iota

                @plsc.parallel_loop(0, N_GROUPS, unroll=16,
                                    carry=jnp.zeros((16,), jnp.int32))
                def qc(g, qcv):
                    d = db[pl.ds(g * 16, 16)]
                    m = (d >= lo_vec) & (d < hi_vec)
                    return qcv + jnp.where(m, 1, 0).astype(jnp.int32)

                s = qc
                for sh in (1, 2, 4, 8):
                    pbuf[pl.ds(16, 16)] = s
                    s = s + plsc.load_gather(pbuf, [iota + (16 - sh)])
                excl = s - qc
                total = s[15]

                @plsc.parallel_loop(
                    0, N_GROUPS, unroll=8,
                    carry=jnp.full((16,), cnt, jnp.int32) + excl)
                def _fill(g, wp):
                    d = db[pl.ds(g * 16, 16)]
                    sv = sb[pl.ds(g * 16, 16)]
                    m = (d >= lo_vec) & (d < hi_vec)
                    dl = d - lo_vec
                    packed = (base_vec + g * 16) | jax.lax.shift_left(dl, 20)
                    dest = jnp.where(m, wp, CAP - 16 + iota)
                    plsc.store_scatter(svacc, [dest], sv)
                    plsc.store_scatter(pkacc, [dest], packed)
                    return wp + jnp.where(m, 1, 0).astype(jnp.int32)
                cnt = cnt + total

                def drain_cond(carry):
                    return carry[0] >= BATCH

                def drain_body(carry):
                    cc2, infl = carry

                    @pl.when(infl == 1)
                    def _():
                        consume_batch()

                    fire_batch()
                    shift_batch()
                    return (cc2 - BATCH, jnp.int32(1))

                cnt, inflight = lax.while_loop(drain_cond, drain_body,
                                               (cnt, inflight))

                @pl.when(c + 2 < N_CHUNKS)
                def _prefetch():
                    start_stage(c + 2, db, sb, semd, sems)
            return (cnt, inflight)

        cnt, inflight = lax.fori_loop(0, N_CHUNKS // 2, chunk_pair_body,
                                      (jnp.int32(0), jnp.int32(0)))

        @pl.when(inflight == 1)
        def _final_consume():
            consume_batch()

        pad_pk = (jnp.full((16,), TRASH << 20, jnp.int32)
                  | (wid * SC_CHUNK + iota))
        for j in range(BATCH // 16):
            dest = jnp.full((16,), cnt, jnp.int32) + iota + j * 16
            plsc.store_scatter(svacc, [dest], lo_vec + iota)
            plsc.store_scatter(pkacc, [dest], pad_pk + j * 16)
        fire_batch()
        consume_batch()

        def out_body(ob, _):
            for rr in range(32):
                r = ob * 32 + rr
                w0 = agg[pl.ds(r * 32, 16)]
                w1 = agg[pl.ds(r * 32 + 16, 16)]
                stg[rr, pl.ds(0, 16)] = plsc.bitcast(
                    jax.lax.shift_left(w0, 16), jnp.float32)
                stg[rr, pl.ds(16, 16)] = plsc.bitcast(w0 & -65536,
                                                      jnp.float32)
                stg[rr, pl.ds(32, 16)] = plsc.bitcast(
                    jax.lax.shift_left(w1, 16), jnp.float32)
                stg[rr, pl.ds(48, 16)] = plsc.bitcast(w1 & -65536,
                                                      jnp.float32)
            pltpu.sync_copy(stg, out_hbm.at[pl.ds(lo + ob * 32, 32), :])
            return 0

        lax.fori_loop(0, R // 32, out_body, 0)


def _sc_agg(h_p, eaw, src_p, dst_p):
    mesh = plsc.VectorSubcoreMesh(core_axis_name="c", subcore_axis_name="s")
    f = pl.kernel(
        _sc_agg_body,
        out_type=jax.ShapeDtypeStruct((N_P, HP), jnp.float32),
        mesh=mesh,
        compiler_params=pltpu.CompilerParams(needs_layout_passes=False),
        scratch_types=[
            pltpu.VMEM((SC_CHUNK,), jnp.int32),
            pltpu.VMEM((SC_CHUNK,), jnp.int32),
            pltpu.VMEM((SC_CHUNK,), jnp.int32),
            pltpu.VMEM((SC_CHUNK,), jnp.int32),
            pltpu.VMEM((CAP,), jnp.int32),
            pltpu.VMEM((CAP,), jnp.int32),
            pltpu.VMEM((BATCH,), jnp.int32),
            pltpu.VMEM((BATCH,), jnp.int32),
            pltpu.VMEM((BATCH,), jnp.int32),
            pltpu.VMEM((32, HP), jnp.float32),
            pltpu.VMEM((32,), jnp.int32),
            pltpu.VMEM((BATCH, HP), jnp.float32),
            pltpu.VMEM((BATCH, HP), jnp.float32),
            pltpu.VMEM(((R + 1) * 32,), jnp.int32),
            pltpu.SemaphoreType.DMA,
            pltpu.SemaphoreType.DMA,
            pltpu.SemaphoreType.DMA,
            pltpu.SemaphoreType.DMA,
            pltpu.SemaphoreType.DMA,
            pltpu.SemaphoreType.DMA,
        ],
    )
    return f(h_p, eaw, src_p, dst_p)


def kernel(x, edge_index, batch, edge_attr, params):
    p = params
    x_p = jnp.pad(x, ((0, N_P - N), (0, 0)))
    batch_p = jnp.pad(batch, (0, N_P - N), constant_values=G)
    batch2 = batch_p.reshape(N_P, 1)

    ea_p = jnp.pad(edge_attr, ((0, E_P - E), (0, 0)))
    src_p = jnp.pad(edge_index[0], (0, E_P - E))
    dst_p = jnp.pad(edge_index[1], (0, E_P - E), constant_values=N_P - 1)

    hb, h_f = _node_mlp(x_p, p)
    eaw = _edge_mlp(ea_p, p)
    agg_p = _sc_agg(hb, eaw, src_p, dst_p)

    o, sig = _head(h_f, agg_p, batch2, p)
    return (o, sig)


# speedup vs baseline: 2.5550x; 1.0003x over previous
import functools

import jax
import jax.numpy as jnp
from jax import lax
from jax.experimental import pallas as pl
from jax.experimental.pallas import tpu as pltpu
from jax.experimental.pallas import tpu_sc as plsc

N = 50000
E = 800000
DIN = 128
DE = 16
H = 64
G = 64
NEG_SLOPE = 0.01

N_P = 50176
HP = 128
NODE_BLK = 1024
N_GRID = N_P // NODE_BLK
EDGE_BLK = 3584
E_GRID = 802816 // EDGE_BLK

PI = tuple((m // 32) * 32 + (m % 32) // 2 + (m % 2) * 16 for m in range(64))


def _leaky(v):
    return jnp.where(v >= 0, v, NEG_SLOPE * v)


def _node_mlp_body(x_ref, w1, b1, w2, b2, w3, b3, out_ref, outp_ref):
    h = jnp.maximum(jnp.dot(x_ref[...], w1[...],
                            preferred_element_type=jnp.float32) + b1[...], 0.0)
    h = jnp.maximum(jnp.dot(h, w2[...],
                            preferred_element_type=jnp.float32) + b2[...], 0.0)
    hp = jnp.dot(h, w3[...], preferred_element_type=jnp.float32) + b3[...]
    out_ref[...] = hp
    outp_ref[...] = hp[:, :H]


def _node_mlp(x_p, p):
    full = lambda shape: pl.BlockSpec(shape, lambda i: (0,) * len(shape))
    return pl.pallas_call(
        _node_mlp_body,
        grid=(N_GRID,),
        in_specs=[
            pl.BlockSpec((NODE_BLK, DIN), lambda i: (i, 0)),
            full((DIN, H)), full((1, H)),
            full((H, H)), full((1, H)),
            full((H, HP)), full((1, HP)),
        ],
        out_specs=[pl.BlockSpec((NODE_BLK, HP), lambda i: (i, 0)),
                   pl.BlockSpec((NODE_BLK, H), lambda i: (i, 0))],
        out_shape=[jax.ShapeDtypeStruct((N_P, HP), jnp.float32),
                   jax.ShapeDtypeStruct((N_P, H), jnp.float32)],
    )(x_p, p['Wnx1'], p['bnx1'].reshape(1, H),
      p['Wnx2'], p['bnx2'].reshape(1, H),
      jnp.pad(p['Wnx3'], ((0, 0), (0, HP - H))),
      jnp.pad(p['bnx3'], (0, HP - H)).reshape(1, HP))


def _edge_mlp_body(ea_ref, w1, b1, w2, b2, w3, b3, we, be, out_ref):
    t = jnp.maximum(jnp.dot(ea_ref[...], w1[...],
                            preferred_element_type=jnp.float32) + b1[...], 0.0)
    t = jnp.maximum(jnp.dot(t, w2[...],
                            preferred_element_type=jnp.float32) + b2[...], 0.0)
    w3e = jnp.dot(w3[...], we[...], preferred_element_type=jnp.float32)
    b3e = jnp.dot(b3[...], we[...], preferred_element_type=jnp.float32) + be[...]
    out_ref[...] = jnp.dot(t, w3e, preferred_element_type=jnp.float32) + b3e


def _edge_mlp(edge_attr, p):
    full = lambda shape: pl.BlockSpec(shape, lambda i: (0,) * len(shape))
    return pl.pallas_call(
        _edge_mlp_body,
        grid=(E_GRID,),
        in_specs=[
            pl.BlockSpec((EDGE_BLK, DE), lambda i: (i, 0)),
            full((DE, H)), full((1, H)),
            full((H, H)), full((1, H)),
            full((H, H)), full((1, H)),
            full((H, HP)), full((1, HP)),
        ],
        out_specs=pl.BlockSpec((EDGE_BLK, HP), lambda i: (i, 0)),
        out_shape=jax.ShapeDtypeStruct((E_P, HP), jnp.float32),
    )(edge_attr, p['Wne1'], p['bne1'].reshape(1, H),
      p['Wne2'], p['bne2'].reshape(1, H),
      p['Wne3'], p['bne3'].reshape(1, H),
      jnp.pad(p['We'], ((0, 0), (0, HP - H))),
      jnp.pad(p['be'], (0, HP - H)).reshape(1, HP))


def _head_body(h_ref, agg_ref, ids_ref, eps_ref, wg1, bg1, wg2, bg2,
               wo1, bo1, gamma, beta, wo2, bo2,
               o_ref, sig_ref, hp_ref):
    step = pl.program_id(0)

    @pl.when(step == 0)
    def _init():
        hp_ref[...] = jnp.full((G, H), -1e30, jnp.float32)

    z = (1.0 + eps_ref[0, 0]) * h_ref[...] + agg_ref[:, :H]
    z = _leaky(jnp.dot(z, wg1[...], preferred_element_type=jnp.float32)
               + bg1[...])
    z2 = jnp.dot(z, wg2[...], preferred_element_type=jnp.float32) + bg2[...]

    ids = ids_ref[...]
    gmin = jnp.min(ids)
    gmax = jnp.minimum(jnp.max(ids), G - 1)

    def body(g, _):
        mask = ids == g
        m = jnp.max(jnp.where(mask, z2, -1e30), axis=0, keepdims=True)
        cur = hp_ref[pl.ds(g, 1), :]
        hp_ref[pl.ds(g, 1), :] = jnp.maximum(cur, m)
        return 0

    jax.lax.fori_loop(gmin, gmax + 1, body, 0)

    @pl.when(step == N_GRID - 1)
    def _head():
        hp = hp_ref[...]
        hp = jnp.where(hp < -1e29, 0.0, hp)
        o = jnp.dot(hp, wo1[...], preferred_element_type=jnp.float32) + bo1[...]
        o = o * (1.0 / jnp.sqrt(1.0 + 1e-5)) * gamma[...] + beta[...]
        o = _leaky(o)
        o2 = jnp.dot(o, wo2[...], preferred_element_type=jnp.float32) + bo2[0, 0]
        o_ref[...] = o2
        sig_ref[...] = 1.0 / (1.0 + jnp.exp(-o2))


def _head(h_p, agg_p, batch2, p):
    full = lambda shape: pl.BlockSpec(shape, lambda i: (0,) * len(shape))
    return pl.pallas_call(
        _head_body,
        grid=(N_GRID,),
        in_specs=[
            pl.BlockSpec((NODE_BLK, H), lambda i: (i, 0)),
            pl.BlockSpec((NODE_BLK, HP), lambda i: (i, 0)),
            pl.BlockSpec((NODE_BLK, 1), lambda i: (i, 0)),
            full((1, 1)),
            full((H, H)), full((1, H)),
            full((H, H)), full((1, H)),
            full((H, H)), full((1, H)),
            full((1, H)), full((1, H)),
            full((H, 1)), full((1, 1)),
        ],
        out_specs=[full((G, 1)), full((G, 1))],
        out_shape=[jax.ShapeDtypeStruct((G, 1), jnp.float32),
                   jax.ShapeDtypeStruct((G, 1), jnp.float32)],
        scratch_shapes=[pltpu.VMEM((G, H), jnp.float32)],
    )(h_p, agg_p, batch2, p['eps'].reshape(1, 1),
      p['Wg1'], p['bg1'].reshape(1, H),
      p['Wg2'], p['bg2'].reshape(1, H),
      p['Wo1'], p['bo1'].reshape(1, H),
      p['gamma'].reshape(1, H), p['beta'].reshape(1, H),
      p['Wo2'], p['bo2'].reshape(1, 1))


NW = 32
R = N_P // NW
TRASH = R
SC_CHUNK = 4096
N_GROUPS = SC_CHUNK // 16
E_P = 802816
N_CHUNKS = E_P // SC_CHUNK
BATCH = 128
CAP = BATCH + SC_CHUNK + 16


def _sc_agg_body(h_hbm, eaw_hbm, src_hbm, dst_hbm, out_hbm,
                 dstv0, dstv1, srcv0, srcv1, svacc, pkacc, idbuf, dlbuf,
                 gsv, stg, pbuf, hrows, erows, agg, sem1, sem2,
                 sd0, sd1, ss0, ss1):
    cid = lax.axis_index("c")
    sid = lax.axis_index("s")
    wid = sid * 2 + cid
    iota = lax.iota(jnp.int32, 16)
    zero16 = jnp.zeros((16,), jnp.float32)
    pbuf[pl.ds(0, 16)] = jnp.zeros((16,), jnp.int32)

    def fire_batch():
        @plsc.parallel_loop(0, BATCH // 16, unroll=4)
        def _unpack(g):
            v = pkacc[pl.ds(g * 16, 16)]
            idbuf[pl.ds(g * 16, 16)] = v & 0xFFFFF
            dlbuf[pl.ds(g * 16, 16)] = jax.lax.shift_right_logical(v, 20)
            gsv[pl.ds(g * 16, 16)] = svacc[pl.ds(g * 16, 16)]
        pltpu.async_copy(h_hbm.at[gsv], hrows, sem1)
        pltpu.async_copy(eaw_hbm.at[idbuf], erows, sem2)

    def consume_batch():
        pltpu.make_async_copy(h_hbm.at[gsv], hrows, sem1).wait()
        pltpu.make_async_copy(eaw_hbm.at[idbuf], erows, sem2).wait()

        def edge_grp_body(g, _):
            dlv = dlbuf[pl.ds(g * 16, 16)] * 32
            for lane in range(16):
                i = g * 16 + lane
                rb = dlv[lane]
                for q in range(2):
                    hv0 = hrows[i, pl.ds(q * 32, 16)]
                    hv1 = hrows[i, pl.ds(q * 32 + 16, 16)]
                    ev0 = erows[i, pl.ds(q * 32, 16)]
                    ev1 = erows[i, pl.ds(q * 32 + 16, 16)]
                    m0 = jnp.maximum(hv0 + ev0, 0.0)
                    m1 = jnp.maximum(hv1 + ev1, 0.0)
                    mb0 = jax.lax.shift_right_logical(
                        plsc.bitcast(m0, jnp.int32) + 0x8000, 16)
                    mb1 = jax.lax.shift_right_logical(
                        plsc.bitcast(m1, jnp.int32) + 0x8000, 16)
                    cur = agg[pl.ds(rb + q * 16, 16)]
                    nlo = jnp.maximum(cur & 0xFFFF, mb0)
                    nhi = jnp.maximum(cur & -65536,
                                      jax.lax.shift_left(mb1, 16))
                    agg[pl.ds(rb + q * 16, 16)] = nlo | nhi
            return 0

        lax.fori_loop(0, BATCH // 16, edge_grp_body, 0)

    def shift_batch():
        def shift_body(j, _):
            for ref in (svacc, pkacc):
                ref[pl.ds(j * 16, 16)] = ref[pl.ds(BATCH + j * 16, 16)]
            return 0

        lax.fori_loop(0, SC_CHUNK // 16, shift_body, 0)

    if True:
        lo = wid * R
        hi = lo + R
        lo_vec = jnp.full((16,), lo, jnp.int32)
        hi_vec = jnp.full((16,), hi, jnp.int32)
        zero16i = jnp.zeros((16,), jnp.int32)

        @plsc.parallel_loop(0, R + 1, unroll=8)
        def _zero(r):
            for q in range(2):
                agg[pl.ds(r * 32 + q * 16, 16)] = zero16i

        def start_stage(c, db, sb, semd, sems):
            base = c * SC_CHUNK
            pltpu.async_copy(dst_hbm.at[pl.ds(base, SC_CHUNK)], db, semd)
            pltpu.async_copy(src_hbm.at[pl.ds(base, SC_CHUNK)], sb, sems)

        start_stage(0, dstv0, srcv0, sd0, ss0)
        start_stage(1, dstv1, srcv1, sd1, ss1)

        def chunk_pair_body(cc, carry):
            cnt, inflight = carry
            for b, (db, sb, semd, sems) in enumerate(
                    ((dstv0, srcv0, sd0, ss0), (dstv1, srcv1, sd1, ss1))):
                c = cc * 2 + b
                base = c * SC_CHUNK
                pltpu.make_async_copy(dst_hbm.at[pl.ds(0, SC_CHUNK)],
                                      db, semd).wait()
                pltpu.make_async_copy(src_hbm.at[pl.ds(0, SC_CHUNK)],
                                      sb, sems).wait()
                base_vec = jnp.full((16,), base, jnp.int32) + iota

                @plsc.parallel_loop(0, N_GROUPS, unroll=16,
                                    carry=jnp.zeros((16,), jnp.int32))
                def qc(g, qcv):
                    d = db[pl.ds(g * 16, 16)]
                    m = (d >= lo_vec) & (d < hi_vec)
                    return qcv + jnp.where(m, 1, 0).astype(jnp.int32)

                s = qc
                for sh in (1, 2, 4, 8):
                    pbuf[pl.ds(16, 16)] = s
                    s = s + plsc.load_gather(pbuf, [iota + (16 - sh)])
                excl = s - qc
                total = s[15]

                @plsc.parallel_loop(
                    0, N_GROUPS, unroll=8,
                    carry=jnp.full((16,), cnt, jnp.int32) + excl)
                def _fill(g, wp):
                    d = db[pl.ds(g * 16, 16)]
                    sv = sb[pl.ds(g * 16, 16)]
                    m = (d >= lo_vec) & (d < hi_vec)
                    dl = d - lo_vec
                    packed = (base_vec + g * 16) | jax.lax.shift_left(dl, 20)
                    dest = jnp.where(m, wp, CAP - 16 + iota)
                    plsc.store_scatter(svacc, [dest], sv)
                    plsc.store_scatter(pkacc, [dest], packed)
                    return wp + jnp.where(m, 1, 0).astype(jnp.int32)
                cnt = cnt + total

                def drain_cond(carry):
                    return carry[0] >= BATCH

                def drain_body(carry):
                    cc2, infl = carry

                    @pl.when(infl == 1)
                    def _():
                        consume_batch()

                    fire_batch()
                    shift_batch()
                    return (cc2 - BATCH, jnp.int32(1))

                cnt, inflight = lax.while_loop(drain_cond, drain_body,
                                               (cnt, inflight))

                @pl.when(c + 2 < N_CHUNKS)
                def _prefetch():
                    start_stage(c + 2, db, sb, semd, sems)
            return (cnt, inflight)

        cnt, inflight = lax.fori_loop(0, N_CHUNKS // 2, chunk_pair_body,
                                      (jnp.int32(0), jnp.int32(0)))

        @pl.when(inflight == 1)
        def _final_consume():
            consume_batch()

        pad_pk = (jnp.full((16,), TRASH << 20, jnp.int32)
                  | (wid * SC_CHUNK + iota))
        for j in range(BATCH // 16):
            dest = jnp.full((16,), cnt, jnp.int32) + iota + j * 16
            plsc.store_scatter(svacc, [dest], lo_vec + iota)
            plsc.store_scatter(pkacc, [dest], pad_pk + j * 16)
        fire_batch()
        consume_batch()

        def out_body(ob, _):
            for rr in range(32):
                r = ob * 32 + rr
                w0 = agg[pl.ds(r * 32, 16)]
                w1 = agg[pl.ds(r * 32 + 16, 16)]
                stg[rr, pl.ds(0, 16)] = plsc.bitcast(
                    jax.lax.shift_left(w0, 16), jnp.float32)
                stg[rr, pl.ds(16, 16)] = plsc.bitcast(w0 & -65536,
                                                      jnp.float32)
                stg[rr, pl.ds(32, 16)] = plsc.bitcast(
                    jax.lax.shift_left(w1, 16), jnp.float32)
                stg[rr, pl.ds(48, 16)] = plsc.bitcast(w1 & -65536,
                                                      jnp.float32)
            pltpu.sync_copy(stg, out_hbm.at[pl.ds(lo + ob * 32, 32), :])
            return 0

        lax.fori_loop(0, R // 32, out_body, 0)


def _sc_agg(h_p, eaw, src_p, dst_p):
    mesh = plsc.VectorSubcoreMesh(core_axis_name="c", subcore_axis_name="s")
    f = pl.kernel(
        _sc_agg_body,
        out_type=jax.ShapeDtypeStruct((N_P, HP), jnp.float32),
        mesh=mesh,
        compiler_params=pltpu.CompilerParams(needs_layout_passes=False),
        scratch_types=[
            pltpu.VMEM((SC_CHUNK,), jnp.int32),
            pltpu.VMEM((SC_CHUNK,), jnp.int32),
            pltpu.VMEM((SC_CHUNK,), jnp.int32),
            pltpu.VMEM((SC_CHUNK,), jnp.int32),
            pltpu.VMEM((CAP,), jnp.int32),
            pltpu.VMEM((CAP,), jnp.int32),
            pltpu.VMEM((BATCH,), jnp.int32),
            pltpu.VMEM((BATCH,), jnp.int32),
            pltpu.VMEM((BATCH,), jnp.int32),
            pltpu.VMEM((32, HP), jnp.float32),
            pltpu.VMEM((32,), jnp.int32),
            pltpu.VMEM((BATCH, HP), jnp.float32),
            pltpu.VMEM((BATCH, HP), jnp.float32),
            pltpu.VMEM(((R + 1) * 32,), jnp.int32),
            pltpu.SemaphoreType.DMA,
            pltpu.SemaphoreType.DMA,
            pltpu.SemaphoreType.DMA,
            pltpu.SemaphoreType.DMA,
            pltpu.SemaphoreType.DMA,
            pltpu.SemaphoreType.DMA,
        ],
    )
    return f(h_p, eaw, src_p, dst_p)


def kernel(x, edge_index, batch, edge_attr, params):
    p = params
    x_p = jnp.pad(x, ((0, N_P - N), (0, 0)))
    batch_p = jnp.pad(batch, (0, N_P - N), constant_values=G)
    batch2 = batch_p.reshape(N_P, 1)

    ea_p = jnp.pad(edge_attr, ((0, E_P - E), (0, 0)))
    src_p = jnp.pad(edge_index[0], (0, E_P - E))
    dst_p = jnp.pad(edge_index[1], (0, E_P - E), constant_values=N_P - 1)

    hb, h_f = _node_mlp(x_p, p)
    eaw = _edge_mlp(ea_p, p)
    agg_p = _sc_agg(hb, eaw, src_p, dst_p)

    o, sig = _head(h_f, agg_p, batch2, p)
    return (o, sig)


# speedup vs baseline: 2.5556x; 1.0002x over previous
import jax
import jax.numpy as jnp
from jax import lax
from jax.experimental import pallas as pl
from jax.experimental.pallas import tpu as pltpu
from jax.experimental.pallas import tpu_sc as plsc

N = 50000
E = 800000
DIN = 128
DE = 16
H = 64
G = 64
NEG_SLOPE = 0.01

N_P = 50176
HP = 128
NODE_BLK = 1024
N_GRID = N_P // NODE_BLK
EDGE_BLK = 3584
E_GRID = 802816 // EDGE_BLK


def _leaky(v):
    return jnp.where(v >= 0, v, NEG_SLOPE * v)


def _node_mlp_body(x_ref, w1, b1, w2, b2, w3, b3, out_ref, outp_ref):
    h = jnp.maximum(jnp.dot(x_ref[...], w1[...],
                            preferred_element_type=jnp.float32) + b1[...], 0.0)
    h = jnp.maximum(jnp.dot(h, w2[...],
                            preferred_element_type=jnp.float32) + b2[...], 0.0)
    hp = jnp.dot(h, w3[...], preferred_element_type=jnp.float32) + b3[...]
    out_ref[...] = hp
    outp_ref[...] = hp[:, :H]


def _node_mlp(x_p, p):
    full = lambda shape: pl.BlockSpec(shape, lambda i: (0,) * len(shape))
    return pl.pallas_call(
        _node_mlp_body,
        grid=(N_GRID,),
        in_specs=[
            pl.BlockSpec((NODE_BLK, DIN), lambda i: (i, 0)),
            full((DIN, H)), full((1, H)),
            full((H, H)), full((1, H)),
            full((H, HP)), full((1, HP)),
        ],
        out_specs=[pl.BlockSpec((NODE_BLK, HP), lambda i: (i, 0)),
                   pl.BlockSpec((NODE_BLK, H), lambda i: (i, 0))],
        out_shape=[jax.ShapeDtypeStruct((N_P, HP), jnp.float32),
                   jax.ShapeDtypeStruct((N_P, H), jnp.float32)],
    )(x_p, p['Wnx1'], p['bnx1'].reshape(1, H),
      p['Wnx2'], p['bnx2'].reshape(1, H),
      jnp.pad(p['Wnx3'], ((0, 0), (0, HP - H))),
      jnp.pad(p['bnx3'], (0, HP - H)).reshape(1, HP))


def _edge_mlp_body(ea_ref, w1, b1, w2, b2, w3, b3, we, be, out_ref):
    t = jnp.maximum(jnp.dot(ea_ref[...], w1[...],
                            preferred_element_type=jnp.float32) + b1[...], 0.0)
    t = jnp.maximum(jnp.dot(t, w2[...],
                            preferred_element_type=jnp.float32) + b2[...], 0.0)
    w3e = jnp.dot(w3[...], we[...], preferred_element_type=jnp.float32)
    b3e = jnp.dot(b3[...], we[...], preferred_element_type=jnp.float32) + be[...]
    out_ref[...] = jnp.dot(t, w3e, preferred_element_type=jnp.float32) + b3e


def _edge_mlp(edge_attr, p):
    full = lambda shape: pl.BlockSpec(shape, lambda i: (0,) * len(shape))
    return pl.pallas_call(
        _edge_mlp_body,
        grid=(E_GRID,),
        in_specs=[
            pl.BlockSpec((EDGE_BLK, DE), lambda i: (i, 0)),
            full((DE, H)), full((1, H)),
            full((H, H)), full((1, H)),
            full((H, H)), full((1, H)),
            full((H, HP)), full((1, HP)),
        ],
        out_specs=pl.BlockSpec((EDGE_BLK, HP), lambda i: (i, 0)),
        out_shape=jax.ShapeDtypeStruct((E_P, HP), jnp.float32),
    )(edge_attr, p['Wne1'], p['bne1'].reshape(1, H),
      p['Wne2'], p['bne2'].reshape(1, H),
      p['Wne3'], p['bne3'].reshape(1, H),
      jnp.pad(p['We'], ((0, 0), (0, HP - H))),
      jnp.pad(p['be'], (0, HP - H)).reshape(1, HP))


def _head_body(h_ref, agg_ref, ids_ref, eps_ref, wg1, bg1, wg2, bg2,
               wo1, bo1, gamma, beta, wo2, bo2,
               o_ref, sig_ref, hp_ref):
    step = pl.program_id(0)

    @pl.when(step == 0)
    def _init():
        hp_ref[...] = jnp.full((G, H), -1e30, jnp.float32)

    z = (1.0 + eps_ref[0, 0]) * h_ref[...] + agg_ref[:, :H]
    z = _leaky(jnp.dot(z, wg1[...], preferred_element_type=jnp.float32)
               + bg1[...])
    z2 = jnp.dot(z, wg2[...], preferred_element_type=jnp.float32) + bg2[...]

    ids = ids_ref[...]
    gmin = jnp.min(ids)
    gmax = jnp.minimum(jnp.max(ids), G - 1)

    def body(g, _):
        mask = ids == g
        m = jnp.max(jnp.where(mask, z2, -1e30), axis=0, keepdims=True)
        cur = hp_ref[pl.ds(g, 1), :]
        hp_ref[pl.ds(g, 1), :] = jnp.maximum(cur, m)
        return 0

    jax.lax.fori_loop(gmin, gmax + 1, body, 0)

    @pl.when(step == N_GRID - 1)
    def _head():
        hp = hp_ref[...]
        hp = jnp.where(hp < -1e29, 0.0, hp)
        o = jnp.dot(hp, wo1[...], preferred_element_type=jnp.float32) + bo1[...]
        o = o * (1.0 / jnp.sqrt(1.0 + 1e-5)) * gamma[...] + beta[...]
        o = _leaky(o)
        o2 = jnp.dot(o, wo2[...], preferred_element_type=jnp.float32) + bo2[0, 0]
        o_ref[...] = o2
        sig_ref[...] = 1.0 / (1.0 + jnp.exp(-o2))


def _head(h_p, agg_p, batch2, p):
    full = lambda shape: pl.BlockSpec(shape, lambda i: (0,) * len(shape))
    return pl.pallas_call(
        _head_body,
        grid=(N_GRID,),
        in_specs=[
            pl.BlockSpec((NODE_BLK, H), lambda i: (i, 0)),
            pl.BlockSpec((NODE_BLK, HP), lambda i: (i, 0)),
            pl.BlockSpec((NODE_BLK, 1), lambda i: (i, 0)),
            full((1, 1)),
            full((H, H)), full((1, H)),
            full((H, H)), full((1, H)),
            full((H, H)), full((1, H)),
            full((1, H)), full((1, H)),
            full((H, 1)), full((1, 1)),
        ],
        out_specs=[full((G, 1)), full((G, 1))],
        out_shape=[jax.ShapeDtypeStruct((G, 1), jnp.float32),
                   jax.ShapeDtypeStruct((G, 1), jnp.float32)],
        scratch_shapes=[pltpu.VMEM((G, H), jnp.float32)],
    )(h_p, agg_p, batch2, p['eps'].reshape(1, 1),
      p['Wg1'], p['bg1'].reshape(1, H),
      p['Wg2'], p['bg2'].reshape(1, H),
      p['Wo1'], p['bo1'].reshape(1, H),
      p['gamma'].reshape(1, H), p['beta'].reshape(1, H),
      p['Wo2'], p['bo2'].reshape(1, 1))


NW = 32
R = N_P // NW
TRASH = R
SC_CHUNK = 4096
N_GROUPS = SC_CHUNK // 16
E_P = 802816
N_CHUNKS = E_P // SC_CHUNK
BATCH = 128
CAP = BATCH + SC_CHUNK + 16


def _sc_agg_body(h_hbm, eaw_hbm, src_hbm, dst_hbm, out_hbm,
                 dstv0, dstv1, srcv0, srcv1, svacc, pkacc, idbuf, dlbuf,
                 gsv, stg, pbuf, hrows, erows, agg, sem1, sem2,
                 sd0, sd1, ss0, ss1):
    cid = lax.axis_index("c")
    sid = lax.axis_index("s")
    wid = sid * 2 + cid
    iota = lax.iota(jnp.int32, 16)
    pbuf[pl.ds(0, 16)] = jnp.zeros((16,), jnp.int32)

    def fire_batch():
        @plsc.parallel_loop(0, BATCH // 16, unroll=4)
        def _unpack(g):
            v = pkacc[pl.ds(g * 16, 16)]
            idbuf[pl.ds(g * 16, 16)] = v & 0xFFFFF
            dlbuf[pl.ds(g * 16, 16)] = jax.lax.shift_right_logical(v, 20)
            gsv[pl.ds(g * 16, 16)] = svacc[pl.ds(g * 16, 16)]
        pltpu.async_copy(h_hbm.at[gsv], hrows, sem1)
        pltpu.async_copy(eaw_hbm.at[idbuf], erows, sem2)

    def consume_batch():
        pltpu.make_async_copy(h_hbm.at[gsv], hrows, sem1).wait()
        pltpu.make_async_copy(eaw_hbm.at[idbuf], erows, sem2).wait()

        def edge_grp_body(g, _):
            dlv = dlbuf[pl.ds(g * 16, 16)] * 32
            for lane in range(16):
                i = g * 16 + lane
                rb = dlv[lane]
                for q in range(2):
                    hv0 = hrows[i, pl.ds(q * 32, 16)]
                    hv1 = hrows[i, pl.ds(q * 32 + 16, 16)]
                    ev0 = erows[i, pl.ds(q * 32, 16)]
                    ev1 = erows[i, pl.ds(q * 32 + 16, 16)]
                    m0 = jnp.maximum(hv0 + ev0, 0.0)
                    m1 = jnp.maximum(hv1 + ev1, 0.0)
                    mb0 = jax.lax.shift_right_logical(
                        plsc.bitcast(m0, jnp.int32) + 0x8000, 16)
                    mb1 = jax.lax.shift_right_logical(
                        plsc.bitcast(m1, jnp.int32) + 0x8000, 16)
                    cur = agg[pl.ds(rb + q * 16, 16)]
                    nlo = jnp.maximum(cur & 0xFFFF, mb0)
                    nhi = jnp.maximum(cur & -65536,
                                      jax.lax.shift_left(mb1, 16))
                    agg[pl.ds(rb + q * 16, 16)] = nlo | nhi
            return 0

        lax.fori_loop(0, BATCH // 16, edge_grp_body, 0)

    def shift_batch():
        def shift_body(j, _):
            for ref in (svacc, pkacc):
                ref[pl.ds(j * 16, 16)] = ref[pl.ds(BATCH + j * 16, 16)]
            return 0

        lax.fori_loop(0, SC_CHUNK // 16, shift_body, 0)

    if True:
        lo = wid * R
        hi = lo + R
        lo_vec = jnp.full((16,), lo, jnp.int32)
        hi_vec = jnp.full((16,), hi, jnp.int32)
        zero16i = jnp.zeros((16,), jnp.int32)

        @plsc.parallel_loop(0, R + 1, unroll=8)
        def _zero(r):
            for q in range(2):
                agg[pl.ds(r * 32 + q * 16, 16)] = zero16i

        def start_stage(c, db, sb, semd, sems):
            base = c * SC_CHUNK
            pltpu.async_copy(dst_hbm.at[pl.ds(base, SC_CHUNK)], db, semd)
            pltpu.async_copy(src_hbm.at[pl.ds(base, SC_CHUNK)], sb, sems)

        start_stage(0, dstv0, srcv0, sd0, ss0)
        start_stage(1, dstv1, srcv1, sd1, ss1)

        def chunk_pair_body(cc, carry):
            cnt, inflight = carry
            for b, (db, sb, semd, sems) in enumerate(
                    ((dstv0, srcv0, sd0, ss0), (dstv1, srcv1, sd1, ss1))):
                c = cc * 2 + b
                base = c * SC_CHUNK
                pltpu.make_async_copy(dst_hbm.at[pl.ds(0, SC_CHUNK)],
                                      db, semd).wait()
                pltpu.make_async_copy(src_hbm.at[pl.ds(0, SC_CHUNK)],
                                      sb, sems).wait()
                base_vec = jnp.full((16,), base, jnp.int32) + iota

                @plsc.parallel_loop(0, N_GROUPS, unroll=16,
                                    carry=jnp.zeros((16,), jnp.int32))
                def qc(g, qcv):
                    d = db[pl.ds(g * 16, 16)]
                    m = (d >= lo_vec) & (d < hi_vec)
                    return qcv + jnp.where(m, 1, 0).astype(jnp.int32)

                s = qc
                for sh in (1, 2, 4, 8):
                    pbuf[pl.ds(16, 16)] = s
                    s = s + plsc.load_gather(pbuf, [iota + (16 - sh)])
                excl = s - qc
                total = s[15]

                @plsc.parallel_loop(
                    0, N_GROUPS, unroll=8,
                    carry=jnp.full((16,), cnt, jnp.int32) + excl)
                def _fill(g, wp):
                    d = db[pl.ds(g * 16, 16)]
                    sv = sb[pl.ds(g * 16, 16)]
                    m = (d >= lo_vec) & (d < hi_vec)
                    dl = d - lo_vec
                    packed = (base_vec + g * 16) | jax.lax.shift_left(dl, 20)
                    dest = jnp.where(m, wp, CAP - 16 + iota)
                    plsc.store_scatter(svacc, [dest], sv)
                    plsc.store_scatter(pkacc, [dest], packed)
                    return wp + jnp.where(m, 1, 0).astype(jnp.int32)
                cnt = cnt + total

                def drain_cond(carry):
                    return carry[0] >= BATCH

                def drain_body(carry):
                    cc2, infl = carry

                    @pl.when(infl == 1)
                    def _():
                        consume_batch()

                    fire_batch()
                    shift_batch()
                    return (cc2 - BATCH, jnp.int32(1))

                cnt, inflight = lax.while_loop(drain_cond, drain_body,
                                               (cnt, inflight))

                @pl.when(c + 2 < N_CHUNKS)
                def _prefetch():
                    start_stage(c + 2, db, sb, semd, sems)
            return (cnt, inflight)

        cnt, inflight = lax.fori_loop(0, N_CHUNKS // 2, chunk_pair_body,
                                      (jnp.int32(0), jnp.int32(0)))

        @pl.when(inflight == 1)
        def _final_consume():
            consume_batch()

        pad_pk = (jnp.full((16,), TRASH << 20, jnp.int32)
                  | (wid * SC_CHUNK + iota))
        for j in range(BATCH // 16):
            dest = jnp.full((16,), cnt, jnp.int32) + iota + j * 16
            plsc.store_scatter(svacc, [dest], lo_vec + iota)
            plsc.store_scatter(pkacc, [dest], pad_pk + j * 16)
        fire_batch()
        consume_batch()

        def out_body(ob, _):
            for rr in range(32):
                r = ob * 32 + rr
                w0 = agg[pl.ds(r * 32, 16)]
                w1 = agg[pl.ds(r * 32 + 16, 16)]
                stg[rr, pl.ds(0, 16)] = plsc.bitcast(
                    jax.lax.shift_left(w0, 16), jnp.float32)
                stg[rr, pl.ds(16, 16)] = plsc.bitcast(w0 & -65536,
                                                      jnp.float32)
                stg[rr, pl.ds(32, 16)] = plsc.bitcast(
                    jax.lax.shift_left(w1, 16), jnp.float32)
                stg[rr, pl.ds(48, 16)] = plsc.bitcast(w1 & -65536,
                                                      jnp.float32)
            pltpu.sync_copy(stg, out_hbm.at[pl.ds(lo + ob * 32, 32), :])
            return 0

        lax.fori_loop(0, R // 32, out_body, 0)


def _sc_agg(h_p, eaw, src_p, dst_p):
    mesh = plsc.VectorSubcoreMesh(core_axis_name="c", subcore_axis_name="s")
    f = pl.kernel(
        _sc_agg_body,
        out_type=jax.ShapeDtypeStruct((N_P, HP), jnp.float32),
        mesh=mesh,
        compiler_params=pltpu.CompilerParams(needs_layout_passes=False),
        scratch_types=[
            pltpu.VMEM((SC_CHUNK,), jnp.int32),
            pltpu.VMEM((SC_CHUNK,), jnp.int32),
            pltpu.VMEM((SC_CHUNK,), jnp.int32),
            pltpu.VMEM((SC_CHUNK,), jnp.int32),
            pltpu.VMEM((CAP,), jnp.int32),
            pltpu.VMEM((CAP,), jnp.int32),
            pltpu.VMEM((BATCH,), jnp.int32),
            pltpu.VMEM((BATCH,), jnp.int32),
            pltpu.VMEM((BATCH,), jnp.int32),
            pltpu.VMEM((32, HP), jnp.float32),
            pltpu.VMEM((32,), jnp.int32),
            pltpu.VMEM((BATCH, HP), jnp.float32),
            pltpu.VMEM((BATCH, HP), jnp.float32),
            pltpu.VMEM(((R + 1) * 32,), jnp.int32),
            pltpu.SemaphoreType.DMA,
            pltpu.SemaphoreType.DMA,
            pltpu.SemaphoreType.DMA,
            pltpu.SemaphoreType.DMA,
            pltpu.SemaphoreType.DMA,
            pltpu.SemaphoreType.DMA,
        ],
    )
    return f(h_p, eaw, src_p, dst_p)


def kernel(x, edge_index, batch, edge_attr, params):
    p = params
    x_p = jnp.pad(x, ((0, N_P - N), (0, 0)))
    batch_p = jnp.pad(batch, (0, N_P - N), constant_values=G)
    batch2 = batch_p.reshape(N_P, 1)

    ea_p = jnp.pad(edge_attr, ((0, E_P - E), (0, 0)))
    src_p = jnp.pad(edge_index[0], (0, E_P - E))
    dst_p = jnp.pad(edge_index[1], (0, E_P - E), constant_values=N_P - 1)

    hb, h_f = _node_mlp(x_p, p)
    eaw = _edge_mlp(ea_p, p)
    agg_p = _sc_agg(hb, eaw, src_p, dst_p)

    o, sig = _head(h_f, agg_p, batch2, p)
    return (o, sig)
